# Initial kernel scaffold; baseline (speedup 1.0000x reference)
import jax
import jax.numpy as jnp
from jax.experimental import pallas as pl


def kernel(x, edge_index, batch, W1, b1, W2, b2, W3, b3, W4, b4, W5, b5):
    raise NotImplementedError("write your pallas kernel here")


# speedup vs baseline: 9.2236x; 9.2236x over previous
import functools
import jax
import jax.numpy as jnp
from jax import lax
from jax.experimental import pallas as pl
from jax.experimental.pallas import tpu as pltpu
from jax.experimental.pallas import tpu_sc as plsc

_N = 10000
_E = 320000
_G = 16
_DIMS = [128, 64, 32, 16, 8, 128]

_NC = 2
_NS = 16
_NW = _NC * _NS

_CHUNK = 128
_EPW = 10240
_E_PAD = _NW * _EPW
_NCHUNK = _EPW // _CHUNK
_N_PAD = 10240
_RPW = _N_PAD // _NS
_BLK = 512
_NBLK = _N_PAD // _BLK

_mesh = plsc.VectorSubcoreMesh(core_axis_name="c", subcore_axis_name="s")


def _deg_body(dst_hbm, ones_hbm, zr_hbm, out_hbm, accsh, didx, onesv, sem):
    c = lax.axis_index("c")
    s = lax.axis_index("s")
    w = s * _NC + c
    pltpu.sync_copy(zr_hbm, accsh.at[pl.ds(s * _RPW, _RPW)])
    pltpu.sync_copy(ones_hbm, onesv)
    plsc.subcore_barrier()

    def body(i, _):
        off = pl.multiple_of(w * _EPW + i * _CHUNK, _CHUNK)
        pltpu.sync_copy(dst_hbm.at[pl.ds(off, _CHUNK)], didx)
        pltpu.sync_copy(onesv, accsh.at[didx], add=True)
        return _

    lax.fori_loop(0, _NCHUNK, body, None)
    plsc.subcore_barrier()
    off2 = pl.multiple_of(c * _N_PAD + s * _RPW, _RPW)
    pltpu.sync_copy(accsh.at[pl.ds(s * _RPW, _RPW)], out_hbm.at[pl.ds(off2, _RPW)])


_sc_params = pltpu.CompilerParams(use_tc_tiling_on_sc=False)

_deg_kernel = functools.partial(
    pl.kernel,
    out_type=jax.ShapeDtypeStruct((_NC * _N_PAD, 8), jnp.float32),
    mesh=_mesh,
    compiler_params=_sc_params,
    scratch_types=[
        pltpu.VMEM_SHARED((_N_PAD, 8), jnp.float32),
        pltpu.VMEM((_CHUNK,), jnp.int32),
        pltpu.VMEM((_CHUNK, 8), jnp.float32),
        pltpu.SemaphoreType.DMA,
    ],
)(_deg_body)


def _agg_body(gh_hbm, src_hbm, dst_hbm, zr_hbm, out_hbm, accsh, sidx, didx, rows, sem):
    c = lax.axis_index("c")
    s = lax.axis_index("s")
    w = s * _NC + c
    pltpu.sync_copy(zr_hbm, accsh.at[pl.ds(s * _RPW, _RPW)])
    plsc.subcore_barrier()

    def body(i, _):
        off = pl.multiple_of(w * _EPW + i * _CHUNK, _CHUNK)
        pltpu.sync_copy(src_hbm.at[pl.ds(off, _CHUNK)], sidx)
        pltpu.sync_copy(dst_hbm.at[pl.ds(off, _CHUNK)], didx)
        pltpu.async_copy(gh_hbm.at[sidx], rows, sem).wait()
        pltpu.sync_copy(rows, accsh.at[didx], add=True)
        return _

    lax.fori_loop(0, _NCHUNK, body, None)
    plsc.subcore_barrier()
    off2 = pl.multiple_of(c * _N_PAD + s * _RPW, _RPW)
    pltpu.sync_copy(accsh.at[pl.ds(s * _RPW, _RPW)], out_hbm.at[pl.ds(off2, _RPW)])


def _make_agg(dout):
    return functools.partial(
        pl.kernel,
        out_type=jax.ShapeDtypeStruct((_NC * _N_PAD, dout), jnp.float32),
        mesh=_mesh,
        compiler_params=_sc_params,
        scratch_types=[
            pltpu.VMEM_SHARED((_N_PAD, dout), jnp.float32),
            pltpu.VMEM((_CHUNK,), jnp.int32),
            pltpu.VMEM((_CHUNK,), jnp.int32),
            pltpu.VMEM((_CHUNK, dout), jnp.float32),
            pltpu.SemaphoreType.DMA,
        ],
    )(_agg_body)


_agg_kernels = {d: _make_agg(d) for d in (64, 32, 16, 8, 128)}


def _tc0_body(degp_ref, x_ref, w_ref, gh_ref, dinv_ref):
    deg = degp_ref[0, :, 0:1] + degp_ref[1, :, 0:1] + 1.0
    dinv = lax.rsqrt(deg)
    dinv_ref[...] = dinv
    gh_ref[...] = dinv * jnp.dot(x_ref[...], w_ref[...],
                                 preferred_element_type=jnp.float32)


def _tc0(degp, x_pad, w1):
    d1 = _DIMS[1]
    return pl.pallas_call(
        _tc0_body,
        grid=(_NBLK,),
        in_specs=[
            pl.BlockSpec((2, _BLK, 8), lambda i: (0, i, 0)),
            pl.BlockSpec((_BLK, _DIMS[0]), lambda i: (i, 0)),
            pl.BlockSpec((_DIMS[0], d1), lambda i: (0, 0)),
        ],
        out_specs=[
            pl.BlockSpec((_BLK, d1), lambda i: (i, 0)),
            pl.BlockSpec((_BLK, 1), lambda i: (i, 0)),
        ],
        out_shape=[
            jax.ShapeDtypeStruct((_N_PAD, d1), jnp.float32),
            jax.ShapeDtypeStruct((_N_PAD, 1), jnp.float32),
        ],
    )(degp.reshape(2, _N_PAD, 8), x_pad, w1)


def _tcmid_body(a0_ref, a1_ref, gh_ref, dinv_ref, b_ref, w_ref, out_ref):
    i = pl.program_id(0)
    dinv = dinv_ref[...]
    h = jnp.maximum(dinv * (a0_ref[...] + a1_ref[...] + gh_ref[...]) + b_ref[...], 0.0)
    rowid = i * _BLK + lax.broadcasted_iota(jnp.int32, (_BLK, 1), 0)
    h = jnp.where(rowid < _N, h, 0.0)
    out_ref[...] = dinv * jnp.dot(h, w_ref[...], preferred_element_type=jnp.float32)


def _tcmid(accflat, gh, dinv, b2d, wnext, din, dnext):
    return pl.pallas_call(
        _tcmid_body,
        grid=(_NBLK,),
        in_specs=[
            pl.BlockSpec((_BLK, din), lambda i: (i, 0)),
            pl.BlockSpec((_BLK, din), lambda i: (i + _NBLK, 0)),
            pl.BlockSpec((_BLK, din), lambda i: (i, 0)),
            pl.BlockSpec((_BLK, 1), lambda i: (i, 0)),
            pl.BlockSpec((1, din), lambda i: (0, 0)),
            pl.BlockSpec((din, dnext), lambda i: (0, 0)),
        ],
        out_specs=pl.BlockSpec((_BLK, dnext), lambda i: (i, 0)),
        out_shape=jax.ShapeDtypeStruct((_N_PAD, dnext), jnp.float32),
    )(accflat, accflat, gh, dinv, b2d, wnext)


def _tc5_body(a0_ref, a1_ref, gh_ref, dinv_ref, b_ref, batch_ref, out_ref,
              sums_scr, cnt_scr):
    i = pl.program_id(0)

    @pl.when(i == 0)
    def _():
        sums_scr[...] = jnp.zeros_like(sums_scr)
        cnt_scr[...] = jnp.zeros_like(cnt_scr)

    dinv = dinv_ref[...]
    h = jnp.maximum(dinv * (a0_ref[...] + a1_ref[...] + gh_ref[...]) + b_ref[...], 0.0)
    rowid = i * _BLK + lax.broadcasted_iota(jnp.int32, (_BLK, 1), 0)
    h = jnp.where(rowid < _N, h, 0.0)
    gids = lax.broadcasted_iota(jnp.int32, (_BLK, _G), 1)
    mask = (batch_ref[...] == gids).astype(jnp.float32)
    sums_scr[...] += lax.dot_general(mask, h, (((0,), (0,)), ((), ())),
                                     preferred_element_type=jnp.float32)
    ones = jnp.ones((_BLK, 1), jnp.float32)
    cnt_scr[...] += lax.dot_general(mask, ones, (((0,), (0,)), ((), ())),
                                    preferred_element_type=jnp.float32)

    @pl.when(i == _NBLK - 1)
    def _():
        out_ref[...] = sums_scr[...] / jnp.maximum(cnt_scr[...], 1.0)


def _tc5(accflat, gh, dinv, b2d, batch2d):
    d5 = _DIMS[5]
    return pl.pallas_call(
        _tc5_body,
        grid=(_NBLK,),
        in_specs=[
            pl.BlockSpec((_BLK, d5), lambda i: (i, 0)),
            pl.BlockSpec((_BLK, d5), lambda i: (i + _NBLK, 0)),
            pl.BlockSpec((_BLK, d5), lambda i: (i, 0)),
            pl.BlockSpec((_BLK, 1), lambda i: (i, 0)),
            pl.BlockSpec((1, d5), lambda i: (0, 0)),
            pl.BlockSpec((_BLK, 1), lambda i: (i, 0)),
        ],
        out_specs=pl.BlockSpec((_G, d5), lambda i: (0, 0)),
        out_shape=jax.ShapeDtypeStruct((_G, d5), jnp.float32),
        scratch_shapes=[
            pltpu.VMEM((_G, d5), jnp.float32),
            pltpu.VMEM((_G, 1), jnp.float32),
        ],
    )(accflat, accflat, gh, dinv, b2d, batch2d)


def kernel(x, edge_index, batch, W1, b1, W2, b2, W3, b3, W4, b4, W5, b5):
    src = edge_index[0].astype(jnp.int32)
    dst = edge_index[1].astype(jnp.int32)
    pad_e = jnp.full((_E_PAD - _E,), _N, jnp.int32)
    src_p = jnp.concatenate([src, pad_e])
    dst_p = jnp.concatenate([dst, pad_e])

    x_pad = jnp.zeros((_N_PAD, _DIMS[0]), jnp.float32).at[:_N].set(x)
    batch2d = jnp.full((_N_PAD, 1), _G, jnp.int32).at[:_N, 0].set(batch.astype(jnp.int32))

    ones8 = jnp.ones((_CHUNK, 8), jnp.float32)
    zr8 = jnp.zeros((_RPW, 8), jnp.float32)

    degp = _deg_kernel(dst_p, ones8, zr8)
    gh, dinv = _tc0(degp, x_pad, W1)

    params = [(W2, b1), (W3, b2), (W4, b3), (W5, b4)]
    for l in range(1, 5):
        din, dnext = _DIMS[l], _DIMS[l + 1]
        zr = jnp.zeros((_RPW, din), jnp.float32)
        accflat = _agg_kernels[din](gh, src_p, dst_p, zr)
        wnext, b = params[l - 1]
        gh = _tcmid(accflat, gh, dinv, b.reshape(1, din), wnext, din, dnext)

    d5 = _DIMS[5]
    zr = jnp.zeros((_RPW, d5), jnp.float32)
    accflat = _agg_kernels[d5](gh, src_p, dst_p, zr)
    return _tc5(accflat, gh, dinv, b5.reshape(1, d5), batch2d)


# speedup vs baseline: 13.5026x; 1.4639x over previous
import functools
import jax
import jax.numpy as jnp
from jax import lax
from jax.experimental import pallas as pl
from jax.experimental.pallas import tpu as pltpu
from jax.experimental.pallas import tpu_sc as plsc

_N = 10000
_E = 320000
_G = 16
_DIMS = [128, 64, 32, 16, 8, 128]

_NC = 2
_NS = 16
_NW = _NC * _NS

_CHUNK = 128
_EPW = 10240
_E_PAD = _NW * _EPW
_NCHUNK = _EPW // _CHUNK
_N_PAD = 10240
_RPW = _N_PAD // _NS
_BLK = 512
_NBLK = _N_PAD // _BLK

_mesh = plsc.VectorSubcoreMesh(core_axis_name="c", subcore_axis_name="s")


def _deg_body(dst_hbm, ones_hbm, zr_hbm, out_hbm, accsh, didx3, onesv, sem):
    c = lax.axis_index("c")
    s = lax.axis_index("s")
    w = s * _NC + c
    pltpu.sync_copy(zr_hbm, accsh.at[pl.ds(s * _RPW, _RPW)])
    pltpu.sync_copy(ones_hbm, onesv)
    pltpu.sync_copy(dst_hbm.at[w], didx3)
    plsc.subcore_barrier()

    def body(i, _):
        pltpu.sync_copy(onesv, accsh.at[didx3.at[i, 0]], add=True)
        return _

    lax.fori_loop(0, _NCHUNK, body, None)
    plsc.subcore_barrier()
    off2 = pl.multiple_of(c * _N_PAD + s * _RPW, _RPW)
    pltpu.sync_copy(accsh.at[pl.ds(s * _RPW, _RPW)], out_hbm.at[pl.ds(off2, _RPW)])


_sc_params = pltpu.CompilerParams(use_tc_tiling_on_sc=False)

_deg_kernel = functools.partial(
    pl.kernel,
    out_type=jax.ShapeDtypeStruct((_NC * _N_PAD, 8), jnp.float32),
    mesh=_mesh,
    compiler_params=_sc_params,
    scratch_types=[
        pltpu.VMEM_SHARED((_N_PAD, 8), jnp.float32),
        pltpu.VMEM((_NCHUNK, 1, _CHUNK), jnp.int32),
        pltpu.VMEM((_CHUNK, 8), jnp.float32),
        pltpu.SemaphoreType.DMA,
    ],
)(_deg_body)


def _make_agg(dout, chunk):
    nchunk = _EPW // chunk

    def _agg_body(gh_hbm, src_hbm, dst_hbm, zr_hbm, out_hbm, accsh, sidx3,
                  didx3, rows2, sems):
        c = lax.axis_index("c")
        s = lax.axis_index("s")
        w = s * _NC + c
        pltpu.sync_copy(zr_hbm, accsh.at[pl.ds(s * _RPW, _RPW)])
        pltpu.sync_copy(src_hbm.at[w], sidx3)
        pltpu.sync_copy(dst_hbm.at[w], didx3)
        plsc.subcore_barrier()

        pltpu.async_copy(gh_hbm.at[sidx3.at[0, 0]], rows2.at[0], sems.at[0])

        def body(j, _):
            p = lax.rem(j, 2)

            @pl.when(j + 1 < nchunk)
            def _():
                pltpu.async_copy(gh_hbm.at[sidx3.at[j + 1, 0]],
                                 rows2.at[1 - p], sems.at[1 - p])

            pltpu.make_async_copy(gh_hbm.at[sidx3.at[j, 0]], rows2.at[p],
                                  sems.at[p]).wait()
            pltpu.sync_copy(rows2.at[p], accsh.at[didx3.at[j, 0]], add=True)
            return _

        lax.fori_loop(0, nchunk, body, None)
        plsc.subcore_barrier()
        off2 = pl.multiple_of(c * _N_PAD + s * _RPW, _RPW)
        pltpu.sync_copy(accsh.at[pl.ds(s * _RPW, _RPW)],
                        out_hbm.at[pl.ds(off2, _RPW)])

    return functools.partial(
        pl.kernel,
        out_type=jax.ShapeDtypeStruct((_NC * _N_PAD, dout), jnp.float32),
        mesh=_mesh,
        compiler_params=_sc_params,
        scratch_types=[
            pltpu.VMEM_SHARED((_N_PAD, dout), jnp.float32),
            pltpu.VMEM((nchunk, 1, chunk), jnp.int32),
            pltpu.VMEM((nchunk, 1, chunk), jnp.int32),
            pltpu.VMEM((2, chunk, dout), jnp.float32),
            pltpu.SemaphoreType.DMA((2,)),
        ],
    )(_agg_body)


_agg_chunk = {64: 128, 32: 128, 16: 128, 8: 128, 128: 64}
_agg_kernels = {d: _make_agg(d, _agg_chunk[d]) for d in _agg_chunk}


def _tc0_body(degp_ref, x_ref, w_ref, gh_ref, dinv_ref):
    deg = degp_ref[0, :, 0:1] + degp_ref[1, :, 0:1] + 1.0
    dinv = lax.rsqrt(deg)
    dinv_ref[...] = dinv
    gh_ref[...] = dinv * jnp.dot(x_ref[...], w_ref[...],
                                 preferred_element_type=jnp.float32)


def _tc0(degp, x_pad, w1):
    d1 = _DIMS[1]
    return pl.pallas_call(
        _tc0_body,
        grid=(_NBLK,),
        in_specs=[
            pl.BlockSpec((2, _BLK, 8), lambda i: (0, i, 0)),
            pl.BlockSpec((_BLK, _DIMS[0]), lambda i: (i, 0)),
            pl.BlockSpec((_DIMS[0], d1), lambda i: (0, 0)),
        ],
        out_specs=[
            pl.BlockSpec((_BLK, d1), lambda i: (i, 0)),
            pl.BlockSpec((_BLK, 1), lambda i: (i, 0)),
        ],
        out_shape=[
            jax.ShapeDtypeStruct((_N_PAD, d1), jnp.float32),
            jax.ShapeDtypeStruct((_N_PAD, 1), jnp.float32),
        ],
    )(degp.reshape(2, _N_PAD, 8), x_pad, w1)


def _tcmid_body(a0_ref, a1_ref, gh_ref, dinv_ref, b_ref, w_ref, out_ref):
    i = pl.program_id(0)
    dinv = dinv_ref[...]
    h = jnp.maximum(dinv * (a0_ref[...] + a1_ref[...] + gh_ref[...]) + b_ref[...], 0.0)
    rowid = i * _BLK + lax.broadcasted_iota(jnp.int32, (_BLK, 1), 0)
    h = jnp.where(rowid < _N, h, 0.0)
    out_ref[...] = dinv * jnp.dot(h, w_ref[...], preferred_element_type=jnp.float32)


def _tcmid(accflat, gh, dinv, b2d, wnext, din, dnext):
    return pl.pallas_call(
        _tcmid_body,
        grid=(_NBLK,),
        in_specs=[
            pl.BlockSpec((_BLK, din), lambda i: (i, 0)),
            pl.BlockSpec((_BLK, din), lambda i: (i + _NBLK, 0)),
            pl.BlockSpec((_BLK, din), lambda i: (i, 0)),
            pl.BlockSpec((_BLK, 1), lambda i: (i, 0)),
            pl.BlockSpec((1, din), lambda i: (0, 0)),
            pl.BlockSpec((din, dnext), lambda i: (0, 0)),
        ],
        out_specs=pl.BlockSpec((_BLK, dnext), lambda i: (i, 0)),
        out_shape=jax.ShapeDtypeStruct((_N_PAD, dnext), jnp.float32),
    )(accflat, accflat, gh, dinv, b2d, wnext)


def _tc5_body(a0_ref, a1_ref, gh_ref, dinv_ref, b_ref, batch_ref, out_ref,
              sums_scr, cnt_scr):
    i = pl.program_id(0)

    @pl.when(i == 0)
    def _():
        sums_scr[...] = jnp.zeros_like(sums_scr)
        cnt_scr[...] = jnp.zeros_like(cnt_scr)

    dinv = dinv_ref[...]
    h = jnp.maximum(dinv * (a0_ref[...] + a1_ref[...] + gh_ref[...]) + b_ref[...], 0.0)
    rowid = i * _BLK + lax.broadcasted_iota(jnp.int32, (_BLK, 1), 0)
    h = jnp.where(rowid < _N, h, 0.0)
    gids = lax.broadcasted_iota(jnp.int32, (_BLK, _G), 1)
    mask = (batch_ref[...] == gids).astype(jnp.float32)
    sums_scr[...] += lax.dot_general(mask, h, (((0,), (0,)), ((), ())),
                                     preferred_element_type=jnp.float32)
    ones = jnp.ones((_BLK, 1), jnp.float32)
    cnt_scr[...] += lax.dot_general(mask, ones, (((0,), (0,)), ((), ())),
                                    preferred_element_type=jnp.float32)

    @pl.when(i == _NBLK - 1)
    def _():
        out_ref[...] = sums_scr[...] / jnp.maximum(cnt_scr[...], 1.0)


def _tc5(accflat, gh, dinv, b2d, batch2d):
    d5 = _DIMS[5]
    return pl.pallas_call(
        _tc5_body,
        grid=(_NBLK,),
        in_specs=[
            pl.BlockSpec((_BLK, d5), lambda i: (i, 0)),
            pl.BlockSpec((_BLK, d5), lambda i: (i + _NBLK, 0)),
            pl.BlockSpec((_BLK, d5), lambda i: (i, 0)),
            pl.BlockSpec((_BLK, 1), lambda i: (i, 0)),
            pl.BlockSpec((1, d5), lambda i: (0, 0)),
            pl.BlockSpec((_BLK, 1), lambda i: (i, 0)),
        ],
        out_specs=pl.BlockSpec((_G, d5), lambda i: (0, 0)),
        out_shape=jax.ShapeDtypeStruct((_G, d5), jnp.float32),
        scratch_shapes=[
            pltpu.VMEM((_G, d5), jnp.float32),
            pltpu.VMEM((_G, 1), jnp.float32),
        ],
    )(accflat, accflat, gh, dinv, b2d, batch2d)


def kernel(x, edge_index, batch, W1, b1, W2, b2, W3, b3, W4, b4, W5, b5):
    src = edge_index[0].astype(jnp.int32)
    dst = edge_index[1].astype(jnp.int32)
    pad_e = jnp.full((_E_PAD - _E,), _N, jnp.int32)
    src_f = jnp.concatenate([src, pad_e])
    dst_f = jnp.concatenate([dst, pad_e])
    dst_p = dst_f.reshape(_NW, _NCHUNK, 1, _CHUNK)

    def _eidx(flat, chunk):
        return flat.reshape(_NW, _EPW // chunk, 1, chunk)

    x_pad = jnp.zeros((_N_PAD, _DIMS[0]), jnp.float32).at[:_N].set(x)
    batch2d = jnp.full((_N_PAD, 1), _G, jnp.int32).at[:_N, 0].set(batch.astype(jnp.int32))

    ones8 = jnp.ones((_CHUNK, 8), jnp.float32)
    zr8 = jnp.zeros((_RPW, 8), jnp.float32)

    degp = _deg_kernel(dst_p, ones8, zr8)
    gh, dinv = _tc0(degp, x_pad, W1)

    params = [(W2, b1), (W3, b2), (W4, b3), (W5, b4)]
    for l in range(1, 5):
        din, dnext = _DIMS[l], _DIMS[l + 1]
        ch = _agg_chunk[din]
        zr = jnp.zeros((_RPW, din), jnp.float32)
        accflat = _agg_kernels[din](gh, _eidx(src_f, ch), _eidx(dst_f, ch), zr)
        wnext, b = params[l - 1]
        gh = _tcmid(accflat, gh, dinv, b.reshape(1, din), wnext, din, dnext)

    d5 = _DIMS[5]
    ch = _agg_chunk[d5]
    zr = jnp.zeros((_RPW, d5), jnp.float32)
    accflat = _agg_kernels[d5](gh, _eidx(src_f, ch), _eidx(dst_f, ch), zr)
    return _tc5(accflat, gh, dinv, b5.reshape(1, d5), batch2d)


# speedup vs baseline: 13.5968x; 1.0070x over previous
import functools
import jax
import jax.numpy as jnp
from jax import lax
from jax.experimental import pallas as pl
from jax.experimental.pallas import tpu as pltpu
from jax.experimental.pallas import tpu_sc as plsc

_N = 10000
_E = 320000
_G = 16
_DIMS = [128, 64, 32, 16, 8, 128]

_NC = 2
_NS = 16
_NW = _NC * _NS

_CHUNK = 128
_EPW = 10240
_E_PAD = _NW * _EPW
_NCHUNK = _EPW // _CHUNK
_N_PAD = 10240
_RPW = _N_PAD // _NS
_BLK = 512
_NBLK = _N_PAD // _BLK

_mesh = plsc.VectorSubcoreMesh(core_axis_name="c", subcore_axis_name="s")


def _deg_body(dst_hbm, ones_hbm, zr_hbm, out_hbm, accsh, didx3, onesv, sem):
    c = lax.axis_index("c")
    s = lax.axis_index("s")
    w = s * _NC + c
    pltpu.sync_copy(zr_hbm, accsh.at[pl.ds(s * _RPW, _RPW)])
    pltpu.sync_copy(ones_hbm, onesv)
    pltpu.sync_copy(dst_hbm.at[w], didx3)
    plsc.subcore_barrier()

    def body(i, _):
        pltpu.sync_copy(onesv, accsh.at[didx3.at[i, 0]], add=True)
        return _

    lax.fori_loop(0, _NCHUNK, body, None)
    plsc.subcore_barrier()
    off2 = pl.multiple_of(c * _N_PAD + s * _RPW, _RPW)
    pltpu.sync_copy(accsh.at[pl.ds(s * _RPW, _RPW)], out_hbm.at[pl.ds(off2, _RPW)])


_sc_params = pltpu.CompilerParams(use_tc_tiling_on_sc=False)

_deg_kernel = functools.partial(
    pl.kernel,
    out_type=jax.ShapeDtypeStruct((_NC * _N_PAD, 8), jnp.float32),
    mesh=_mesh,
    compiler_params=_sc_params,
    scratch_types=[
        pltpu.VMEM_SHARED((_N_PAD, 8), jnp.float32),
        pltpu.VMEM((_NCHUNK, 1, _CHUNK), jnp.int32),
        pltpu.VMEM((_CHUNK, 8), jnp.float32),
        pltpu.SemaphoreType.DMA,
    ],
)(_deg_body)


def _make_agg(dout, chunk):
    nchunk = _EPW // chunk

    def _agg_body(gh_hbm, src_hbm, dst_hbm, zr_hbm, out_hbm, accsh, sidx3,
                  didx3, rows2, gsems, ssems):
        c = lax.axis_index("c")
        s = lax.axis_index("s")
        w = s * _NC + c
        pltpu.sync_copy(zr_hbm, accsh.at[pl.ds(s * _RPW, _RPW)])
        pltpu.sync_copy(src_hbm.at[w], sidx3)
        pltpu.sync_copy(dst_hbm.at[w], didx3)
        plsc.subcore_barrier()

        pltpu.async_copy(gh_hbm.at[sidx3.at[0, 0]], rows2.at[0], gsems.at[0])
        pltpu.async_copy(gh_hbm.at[sidx3.at[1, 0]], rows2.at[1], gsems.at[1])

        def body(j, _):
            p = lax.rem(j, 3)

            @pl.when(j >= 1)
            def _():
                q = lax.rem(j - 1, 3)
                pltpu.make_async_copy(rows2.at[q],
                                      accsh.at[didx3.at[j - 1, 0]],
                                      ssems.at[q]).wait()

            @pl.when(j + 2 < nchunk)
            def _():
                q = lax.rem(j + 2, 3)
                pltpu.async_copy(gh_hbm.at[sidx3.at[j + 2, 0]], rows2.at[q],
                                 gsems.at[q])

            pltpu.make_async_copy(gh_hbm.at[sidx3.at[j, 0]], rows2.at[p],
                                  gsems.at[p]).wait()
            pltpu.async_copy(rows2.at[p], accsh.at[didx3.at[j, 0]],
                             ssems.at[p], add=True)
            return _

        lax.fori_loop(0, nchunk, body, None)
        q = (nchunk - 1) % 3
        pltpu.make_async_copy(rows2.at[q], accsh.at[didx3.at[nchunk - 1, 0]],
                              ssems.at[q]).wait()
        plsc.subcore_barrier()
        off2 = pl.multiple_of(c * _N_PAD + s * _RPW, _RPW)
        pltpu.sync_copy(accsh.at[pl.ds(s * _RPW, _RPW)],
                        out_hbm.at[pl.ds(off2, _RPW)])

    return functools.partial(
        pl.kernel,
        out_type=jax.ShapeDtypeStruct((_NC * _N_PAD, dout), jnp.float32),
        mesh=_mesh,
        compiler_params=_sc_params,
        scratch_types=[
            pltpu.VMEM_SHARED((_N_PAD, dout), jnp.float32),
            pltpu.VMEM((nchunk, 1, chunk), jnp.int32),
            pltpu.VMEM((nchunk, 1, chunk), jnp.int32),
            pltpu.VMEM((3, chunk, dout), jnp.float32),
            pltpu.SemaphoreType.DMA((3,)),
            pltpu.SemaphoreType.DMA((3,)),
        ],
    )(_agg_body)


_agg_chunk = {64: 128, 32: 128, 16: 128, 8: 128, 128: 64}
_agg_kernels = {d: _make_agg(d, _agg_chunk[d]) for d in _agg_chunk}


def _tc0_body(degp_ref, x_ref, w_ref, gh_ref, dinv_ref):
    deg = degp_ref[0, :, 0:1] + degp_ref[1, :, 0:1] + 1.0
    dinv = lax.rsqrt(deg)
    dinv_ref[...] = dinv
    gh_ref[...] = dinv * jnp.dot(x_ref[...], w_ref[...],
                                 preferred_element_type=jnp.float32)


def _tc0(degp, x_pad, w1):
    d1 = _DIMS[1]
    return pl.pallas_call(
        _tc0_body,
        grid=(_NBLK,),
        in_specs=[
            pl.BlockSpec((2, _BLK, 8), lambda i: (0, i, 0)),
            pl.BlockSpec((_BLK, _DIMS[0]), lambda i: (i, 0)),
            pl.BlockSpec((_DIMS[0], d1), lambda i: (0, 0)),
        ],
        out_specs=[
            pl.BlockSpec((_BLK, d1), lambda i: (i, 0)),
            pl.BlockSpec((_BLK, 1), lambda i: (i, 0)),
        ],
        out_shape=[
            jax.ShapeDtypeStruct((_N_PAD, d1), jnp.float32),
            jax.ShapeDtypeStruct((_N_PAD, 1), jnp.float32),
        ],
    )(degp.reshape(2, _N_PAD, 8), x_pad, w1)


def _tcmid_body(a0_ref, a1_ref, gh_ref, dinv_ref, b_ref, w_ref, out_ref):
    i = pl.program_id(0)
    dinv = dinv_ref[...]
    h = jnp.maximum(dinv * (a0_ref[...] + a1_ref[...] + gh_ref[...]) + b_ref[...], 0.0)
    rowid = i * _BLK + lax.broadcasted_iota(jnp.int32, (_BLK, 1), 0)
    h = jnp.where(rowid < _N, h, 0.0)
    out_ref[...] = dinv * jnp.dot(h, w_ref[...], preferred_element_type=jnp.float32)


def _tcmid(accflat, gh, dinv, b2d, wnext, din, dnext):
    return pl.pallas_call(
        _tcmid_body,
        grid=(_NBLK,),
        in_specs=[
            pl.BlockSpec((_BLK, din), lambda i: (i, 0)),
            pl.BlockSpec((_BLK, din), lambda i: (i + _NBLK, 0)),
            pl.BlockSpec((_BLK, din), lambda i: (i, 0)),
            pl.BlockSpec((_BLK, 1), lambda i: (i, 0)),
            pl.BlockSpec((1, din), lambda i: (0, 0)),
            pl.BlockSpec((din, dnext), lambda i: (0, 0)),
        ],
        out_specs=pl.BlockSpec((_BLK, dnext), lambda i: (i, 0)),
        out_shape=jax.ShapeDtypeStruct((_N_PAD, dnext), jnp.float32),
    )(accflat, accflat, gh, dinv, b2d, wnext)


def _tc5_body(a0_ref, a1_ref, gh_ref, dinv_ref, b_ref, batch_ref, out_ref,
              sums_scr, cnt_scr):
    i = pl.program_id(0)

    @pl.when(i == 0)
    def _():
        sums_scr[...] = jnp.zeros_like(sums_scr)
        cnt_scr[...] = jnp.zeros_like(cnt_scr)

    dinv = dinv_ref[...]
    h = jnp.maximum(dinv * (a0_ref[...] + a1_ref[...] + gh_ref[...]) + b_ref[...], 0.0)
    rowid = i * _BLK + lax.broadcasted_iota(jnp.int32, (_BLK, 1), 0)
    h = jnp.where(rowid < _N, h, 0.0)
    gids = lax.broadcasted_iota(jnp.int32, (_BLK, _G), 1)
    mask = (batch_ref[...] == gids).astype(jnp.float32)
    sums_scr[...] += lax.dot_general(mask, h, (((0,), (0,)), ((), ())),
                                     preferred_element_type=jnp.float32)
    ones = jnp.ones((_BLK, 1), jnp.float32)
    cnt_scr[...] += lax.dot_general(mask, ones, (((0,), (0,)), ((), ())),
                                    preferred_element_type=jnp.float32)

    @pl.when(i == _NBLK - 1)
    def _():
        out_ref[...] = sums_scr[...] / jnp.maximum(cnt_scr[...], 1.0)


def _tc5(accflat, gh, dinv, b2d, batch2d):
    d5 = _DIMS[5]
    return pl.pallas_call(
        _tc5_body,
        grid=(_NBLK,),
        in_specs=[
            pl.BlockSpec((_BLK, d5), lambda i: (i, 0)),
            pl.BlockSpec((_BLK, d5), lambda i: (i + _NBLK, 0)),
            pl.BlockSpec((_BLK, d5), lambda i: (i, 0)),
            pl.BlockSpec((_BLK, 1), lambda i: (i, 0)),
            pl.BlockSpec((1, d5), lambda i: (0, 0)),
            pl.BlockSpec((_BLK, 1), lambda i: (i, 0)),
        ],
        out_specs=pl.BlockSpec((_G, d5), lambda i: (0, 0)),
        out_shape=jax.ShapeDtypeStruct((_G, d5), jnp.float32),
        scratch_shapes=[
            pltpu.VMEM((_G, d5), jnp.float32),
            pltpu.VMEM((_G, 1), jnp.float32),
        ],
    )(accflat, accflat, gh, dinv, b2d, batch2d)


def kernel(x, edge_index, batch, W1, b1, W2, b2, W3, b3, W4, b4, W5, b5):
    src = edge_index[0].astype(jnp.int32)
    dst = edge_index[1].astype(jnp.int32)
    pad_e = jnp.full((_E_PAD - _E,), _N, jnp.int32)
    src_f = jnp.concatenate([src, pad_e])
    dst_f = jnp.concatenate([dst, pad_e])
    dst_p = dst_f.reshape(_NW, _NCHUNK, 1, _CHUNK)

    def _eidx(flat, chunk):
        return flat.reshape(_NW, _EPW // chunk, 1, chunk)

    x_pad = jnp.zeros((_N_PAD, _DIMS[0]), jnp.float32).at[:_N].set(x)
    batch2d = jnp.full((_N_PAD, 1), _G, jnp.int32).at[:_N, 0].set(batch.astype(jnp.int32))

    ones8 = jnp.ones((_CHUNK, 8), jnp.float32)
    zr8 = jnp.zeros((_RPW, 8), jnp.float32)

    degp = _deg_kernel(dst_p, ones8, zr8)
    gh, dinv = _tc0(degp, x_pad, W1)

    params = [(W2, b1), (W3, b2), (W4, b3), (W5, b4)]
    for l in range(1, 5):
        din, dnext = _DIMS[l], _DIMS[l + 1]
        ch = _agg_chunk[din]
        zr = jnp.zeros((_RPW, din), jnp.float32)
        accflat = _agg_kernels[din](gh, _eidx(src_f, ch), _eidx(dst_f, ch), zr)
        wnext, b = params[l - 1]
        gh = _tcmid(accflat, gh, dinv, b.reshape(1, din), wnext, din, dnext)

    d5 = _DIMS[5]
    ch = _agg_chunk[d5]
    zr = jnp.zeros((_RPW, d5), jnp.float32)
    accflat = _agg_kernels[d5](gh, _eidx(src_f, ch), _eidx(dst_f, ch), zr)
    return _tc5(accflat, gh, dinv, b5.reshape(1, d5), batch2d)


# speedup vs baseline: 19.3789x; 1.4253x over previous
import functools
import jax
import jax.numpy as jnp
from jax import lax
from jax.experimental import pallas as pl
from jax.experimental.pallas import tpu as pltpu
from jax.experimental.pallas import tpu_sc as plsc

_N = 10000
_E = 320000
_G = 16
_DIMS = [128, 64, 32, 16, 8, 128]

_NC = 2
_NS = 16
_NW = _NC * _NS

_CHUNK = 128
_EPW = 10240
_E_PAD = _NW * _EPW
_NCHUNK = _EPW // _CHUNK
_N_PAD = 10240
_RPW = _N_PAD // _NS
_BLK = 512
_NBLK = _N_PAD // _BLK

_mesh = plsc.VectorSubcoreMesh(core_axis_name="c", subcore_axis_name="s")


def _deg_body(dst_hbm, ones_hbm, zr_hbm, out_hbm, accsh, didx3, onesv, sem):
    c = lax.axis_index("c")
    s = lax.axis_index("s")
    w = s * _NC + c
    pltpu.sync_copy(zr_hbm, accsh.at[pl.ds(s * _RPW, _RPW)])
    pltpu.sync_copy(ones_hbm, onesv)
    pltpu.sync_copy(dst_hbm.at[w], didx3)
    plsc.subcore_barrier()

    def body(i, _):
        pltpu.sync_copy(onesv, accsh.at[didx3.at[i, 0]], add=True)
        return _

    lax.fori_loop(0, _NCHUNK, body, None)
    plsc.subcore_barrier()
    off2 = pl.multiple_of(c * _N_PAD + s * _RPW, _RPW)
    pltpu.sync_copy(accsh.at[pl.ds(s * _RPW, _RPW)], out_hbm.at[pl.ds(off2, _RPW)])


_sc_params = pltpu.CompilerParams(use_tc_tiling_on_sc=False)

_deg_kernel = functools.partial(
    pl.kernel,
    out_type=jax.ShapeDtypeStruct((_NC * _N_PAD, 8), jnp.float32),
    mesh=_mesh,
    compiler_params=_sc_params,
    scratch_types=[
        pltpu.VMEM_SHARED((_N_PAD, 8), jnp.float32),
        pltpu.VMEM((_NCHUNK, 1, _CHUNK), jnp.int32),
        pltpu.VMEM((_CHUNK, 8), jnp.float32),
        pltpu.SemaphoreType.DMA,
    ],
)(_deg_body)


def _make_agg(dout, chunk):
    nchunk = _EPW // chunk
    stage = dout <= 64

    def _agg_body(gh_hbm, src_hbm, dst_hbm, zr_hbm, out_hbm, accsh, sidx3,
                  didx3, rows2, gsems, ssems, *maybe_ghs):
        c = lax.axis_index("c")
        s = lax.axis_index("s")
        w = s * _NC + c
        pltpu.sync_copy(zr_hbm, accsh.at[pl.ds(s * _RPW, _RPW)])
        pltpu.sync_copy(src_hbm.at[w], sidx3)
        pltpu.sync_copy(dst_hbm.at[w], didx3)
        if stage:
            ghs = maybe_ghs[0]
            pltpu.sync_copy(gh_hbm.at[pl.ds(s * _RPW, _RPW)],
                            ghs.at[pl.ds(s * _RPW, _RPW)])
            gh_src = ghs
        else:
            gh_src = gh_hbm
        plsc.subcore_barrier()

        pltpu.async_copy(gh_src.at[sidx3.at[0, 0]], rows2.at[0], gsems.at[0])
        pltpu.async_copy(gh_src.at[sidx3.at[1, 0]], rows2.at[1], gsems.at[1])

        def body(j, _):
            p = lax.rem(j, 3)

            @pl.when(j >= 1)
            def _():
                q = lax.rem(j - 1, 3)
                pltpu.make_async_copy(rows2.at[q],
                                      accsh.at[didx3.at[j - 1, 0]],
                                      ssems.at[q]).wait()

            @pl.when(j + 2 < nchunk)
            def _():
                q = lax.rem(j + 2, 3)
                pltpu.async_copy(gh_src.at[sidx3.at[j + 2, 0]], rows2.at[q],
                                 gsems.at[q])

            pltpu.make_async_copy(gh_src.at[sidx3.at[j, 0]], rows2.at[p],
                                  gsems.at[p]).wait()
            pltpu.async_copy(rows2.at[p], accsh.at[didx3.at[j, 0]],
                             ssems.at[p], add=True)
            return _

        lax.fori_loop(0, nchunk, body, None)
        q = (nchunk - 1) % 3
        pltpu.make_async_copy(rows2.at[q], accsh.at[didx3.at[nchunk - 1, 0]],
                              ssems.at[q]).wait()
        plsc.subcore_barrier()
        off2 = pl.multiple_of(c * _N_PAD + s * _RPW, _RPW)
        pltpu.sync_copy(accsh.at[pl.ds(s * _RPW, _RPW)],
                        out_hbm.at[pl.ds(off2, _RPW)])

    return functools.partial(
        pl.kernel,
        out_type=jax.ShapeDtypeStruct((_NC * _N_PAD, dout), jnp.float32),
        mesh=_mesh,
        compiler_params=_sc_params,
        scratch_types=[
            pltpu.VMEM_SHARED((_N_PAD, dout), jnp.float32),
            pltpu.VMEM((nchunk, 1, chunk), jnp.int32),
            pltpu.VMEM((nchunk, 1, chunk), jnp.int32),
            pltpu.VMEM((3, chunk, dout), jnp.float32),
            pltpu.SemaphoreType.DMA((3,)),
            pltpu.SemaphoreType.DMA((3,)),
        ] + ([pltpu.VMEM_SHARED((_N_PAD, dout), jnp.float32)] if stage else []),
    )(_agg_body)


_agg_chunk = {64: 128, 32: 128, 16: 128, 8: 128, 128: 64}
_agg_kernels = {d: _make_agg(d, _agg_chunk[d]) for d in _agg_chunk}


def _tc0_body(degp_ref, x_ref, w_ref, gh_ref, dinv_ref):
    deg = degp_ref[0, :, 0:1] + degp_ref[1, :, 0:1] + 1.0
    dinv = lax.rsqrt(deg)
    dinv_ref[...] = dinv
    gh_ref[...] = dinv * jnp.dot(x_ref[...], w_ref[...],
                                 preferred_element_type=jnp.float32)


def _tc0(degp, x_pad, w1):
    d1 = _DIMS[1]
    return pl.pallas_call(
        _tc0_body,
        grid=(_NBLK,),
        in_specs=[
            pl.BlockSpec((2, _BLK, 8), lambda i: (0, i, 0)),
            pl.BlockSpec((_BLK, _DIMS[0]), lambda i: (i, 0)),
            pl.BlockSpec((_DIMS[0], d1), lambda i: (0, 0)),
        ],
        out_specs=[
            pl.BlockSpec((_BLK, d1), lambda i: (i, 0)),
            pl.BlockSpec((_BLK, 1), lambda i: (i, 0)),
        ],
        out_shape=[
            jax.ShapeDtypeStruct((_N_PAD, d1), jnp.float32),
            jax.ShapeDtypeStruct((_N_PAD, 1), jnp.float32),
        ],
    )(degp.reshape(2, _N_PAD, 8), x_pad, w1)


def _tcmid_body(a0_ref, a1_ref, gh_ref, dinv_ref, b_ref, w_ref, out_ref):
    i = pl.program_id(0)
    dinv = dinv_ref[...]
    h = jnp.maximum(dinv * (a0_ref[...] + a1_ref[...] + gh_ref[...]) + b_ref[...], 0.0)
    rowid = i * _BLK + lax.broadcasted_iota(jnp.int32, (_BLK, 1), 0)
    h = jnp.where(rowid < _N, h, 0.0)
    out_ref[...] = dinv * jnp.dot(h, w_ref[...], preferred_element_type=jnp.float32)


def _tcmid(accflat, gh, dinv, b2d, wnext, din, dnext):
    return pl.pallas_call(
        _tcmid_body,
        grid=(_NBLK,),
        in_specs=[
            pl.BlockSpec((_BLK, din), lambda i: (i, 0)),
            pl.BlockSpec((_BLK, din), lambda i: (i + _NBLK, 0)),
            pl.BlockSpec((_BLK, din), lambda i: (i, 0)),
            pl.BlockSpec((_BLK, 1), lambda i: (i, 0)),
            pl.BlockSpec((1, din), lambda i: (0, 0)),
            pl.BlockSpec((din, dnext), lambda i: (0, 0)),
        ],
        out_specs=pl.BlockSpec((_BLK, dnext), lambda i: (i, 0)),
        out_shape=jax.ShapeDtypeStruct((_N_PAD, dnext), jnp.float32),
    )(accflat, accflat, gh, dinv, b2d, wnext)


def _tc5_body(a0_ref, a1_ref, gh_ref, dinv_ref, b_ref, batch_ref, out_ref,
              sums_scr, cnt_scr):
    i = pl.program_id(0)

    @pl.when(i == 0)
    def _():
        sums_scr[...] = jnp.zeros_like(sums_scr)
        cnt_scr[...] = jnp.zeros_like(cnt_scr)

    dinv = dinv_ref[...]
    h = jnp.maximum(dinv * (a0_ref[...] + a1_ref[...] + gh_ref[...]) + b_ref[...], 0.0)
    rowid = i * _BLK + lax.broadcasted_iota(jnp.int32, (_BLK, 1), 0)
    h = jnp.where(rowid < _N, h, 0.0)
    gids = lax.broadcasted_iota(jnp.int32, (_BLK, _G), 1)
    mask = (batch_ref[...] == gids).astype(jnp.float32)
    sums_scr[...] += lax.dot_general(mask, h, (((0,), (0,)), ((), ())),
                                     preferred_element_type=jnp.float32)
    ones = jnp.ones((_BLK, 1), jnp.float32)
    cnt_scr[...] += lax.dot_general(mask, ones, (((0,), (0,)), ((), ())),
                                    preferred_element_type=jnp.float32)

    @pl.when(i == _NBLK - 1)
    def _():
        out_ref[...] = sums_scr[...] / jnp.maximum(cnt_scr[...], 1.0)


def _tc5(accflat, gh, dinv, b2d, batch2d):
    d5 = _DIMS[5]
    return pl.pallas_call(
        _tc5_body,
        grid=(_NBLK,),
        in_specs=[
            pl.BlockSpec((_BLK, d5), lambda i: (i, 0)),
            pl.BlockSpec((_BLK, d5), lambda i: (i + _NBLK, 0)),
            pl.BlockSpec((_BLK, d5), lambda i: (i, 0)),
            pl.BlockSpec((_BLK, 1), lambda i: (i, 0)),
            pl.BlockSpec((1, d5), lambda i: (0, 0)),
            pl.BlockSpec((_BLK, 1), lambda i: (i, 0)),
        ],
        out_specs=pl.BlockSpec((_G, d5), lambda i: (0, 0)),
        out_shape=jax.ShapeDtypeStruct((_G, d5), jnp.float32),
        scratch_shapes=[
            pltpu.VMEM((_G, d5), jnp.float32),
            pltpu.VMEM((_G, 1), jnp.float32),
        ],
    )(accflat, accflat, gh, dinv, b2d, batch2d)


def kernel(x, edge_index, batch, W1, b1, W2, b2, W3, b3, W4, b4, W5, b5):
    src = edge_index[0].astype(jnp.int32)
    dst = edge_index[1].astype(jnp.int32)
    pad_e = jnp.full((_E_PAD - _E,), _N, jnp.int32)
    src_f = jnp.concatenate([src, pad_e])
    dst_f = jnp.concatenate([dst, pad_e])
    dst_p = dst_f.reshape(_NW, _NCHUNK, 1, _CHUNK)

    def _eidx(flat, chunk):
        return flat.reshape(_NW, _EPW // chunk, 1, chunk)

    x_pad = jnp.zeros((_N_PAD, _DIMS[0]), jnp.float32).at[:_N].set(x)
    batch2d = jnp.full((_N_PAD, 1), _G, jnp.int32).at[:_N, 0].set(batch.astype(jnp.int32))

    ones8 = jnp.ones((_CHUNK, 8), jnp.float32)
    zr8 = jnp.zeros((_RPW, 8), jnp.float32)

    degp = _deg_kernel(dst_p, ones8, zr8)
    gh, dinv = _tc0(degp, x_pad, W1)

    params = [(W2, b1), (W3, b2), (W4, b3), (W5, b4)]
    for l in range(1, 5):
        din, dnext = _DIMS[l], _DIMS[l + 1]
        ch = _agg_chunk[din]
        zr = jnp.zeros((_RPW, din), jnp.float32)
        accflat = _agg_kernels[din](gh, _eidx(src_f, ch), _eidx(dst_f, ch), zr)
        wnext, b = params[l - 1]
        gh = _tcmid(accflat, gh, dinv, b.reshape(1, din), wnext, din, dnext)

    d5 = _DIMS[5]
    ch = _agg_chunk[d5]
    zr = jnp.zeros((_RPW, d5), jnp.float32)
    accflat = _agg_kernels[d5](gh, _eidx(src_f, ch), _eidx(dst_f, ch), zr)
    return _tc5(accflat, gh, dinv, b5.reshape(1, d5), batch2d)


# speedup vs baseline: 27.5080x; 1.4195x over previous
import functools
import jax
import jax.numpy as jnp
from jax import lax
from jax.experimental import pallas as pl
from jax.experimental.pallas import tpu as pltpu
from jax.experimental.pallas import tpu_sc as plsc

_N = 10000
_E = 320000
_G = 16
_DIMS = [128, 64, 32, 16, 8, 128]

_NC = 2
_NS = 16
_NW = _NC * _NS

_CHUNK = 128
_EPW = 10240
_E_PAD = _NW * _EPW
_NCHUNK = _EPW // _CHUNK
_N_PAD = 10240
_RPW = _N_PAD // _NS
_BLK = 512
_NBLK = _N_PAD // _BLK

_mesh = plsc.VectorSubcoreMesh(core_axis_name="c", subcore_axis_name="s")


def _deg_body(dst_hbm, ones_hbm, zr_hbm, out_hbm, accsh, didx3, onesv, sem):
    c = lax.axis_index("c")
    s = lax.axis_index("s")
    w = s * _NC + c
    pltpu.sync_copy(zr_hbm, accsh.at[pl.ds(s * _RPW, _RPW)])
    pltpu.sync_copy(ones_hbm, onesv)
    pltpu.sync_copy(dst_hbm.at[w], didx3)
    plsc.subcore_barrier()

    def body(i, _):
        pltpu.sync_copy(onesv, accsh.at[didx3.at[i, 0]], add=True)
        return _

    lax.fori_loop(0, _NCHUNK, body, None)
    plsc.subcore_barrier()
    off2 = pl.multiple_of(c * _N_PAD + s * _RPW, _RPW)
    pltpu.sync_copy(accsh.at[pl.ds(s * _RPW, _RPW)], out_hbm.at[pl.ds(off2, _RPW)])


_sc_params = pltpu.CompilerParams(use_tc_tiling_on_sc=False)

_deg_kernel = functools.partial(
    pl.kernel,
    out_type=jax.ShapeDtypeStruct((_NC * _N_PAD, 8), jnp.float32),
    mesh=_mesh,
    compiler_params=_sc_params,
    scratch_types=[
        pltpu.VMEM_SHARED((_N_PAD, 8), jnp.float32),
        pltpu.VMEM((_NCHUNK, 1, _CHUNK), jnp.int32),
        pltpu.VMEM((_CHUNK, 8), jnp.float32),
        pltpu.SemaphoreType.DMA,
    ],
)(_deg_body)


def _make_agg(dout, chunk):
    nchunk = _EPW // chunk
    stage = dout <= 64

    def _agg_body(gh_hbm, src_hbm, dst_hbm, zr_hbm, out_hbm, accsh, sidx3,
                  didx3, rows2, gsems, ssems, *maybe_ghs):
        c = lax.axis_index("c")
        s = lax.axis_index("s")
        w = s * _NC + c
        pltpu.sync_copy(zr_hbm, accsh.at[pl.ds(s * _RPW, _RPW)])
        pltpu.sync_copy(src_hbm.at[w], sidx3)
        pltpu.sync_copy(dst_hbm.at[w], didx3)
        if stage:
            ghs = maybe_ghs[0]
            pltpu.sync_copy(gh_hbm.at[pl.ds(s * _RPW, _RPW)],
                            ghs.at[pl.ds(s * _RPW, _RPW)])
            gh_src = ghs
        else:
            gh_src = gh_hbm
        plsc.subcore_barrier()

        pltpu.async_copy(gh_src.at[sidx3.at[0, 0]], rows2.at[0], gsems.at[0])
        pltpu.async_copy(gh_src.at[sidx3.at[1, 0]], rows2.at[1], gsems.at[1])

        def body(j, _):
            p = lax.rem(j, 3)

            @pl.when(j >= 1)
            def _():
                q = lax.rem(j - 1, 3)
                pltpu.make_async_copy(rows2.at[q],
                                      accsh.at[didx3.at[j - 1, 0]],
                                      ssems.at[q]).wait()

            @pl.when(j + 2 < nchunk)
            def _():
                q = lax.rem(j + 2, 3)
                pltpu.async_copy(gh_src.at[sidx3.at[j + 2, 0]], rows2.at[q],
                                 gsems.at[q])

            pltpu.make_async_copy(gh_src.at[sidx3.at[j, 0]], rows2.at[p],
                                  gsems.at[p]).wait()
            pltpu.async_copy(rows2.at[p], accsh.at[didx3.at[j, 0]],
                             ssems.at[p], add=True)
            return _

        lax.fori_loop(0, nchunk, body, None)
        q = (nchunk - 1) % 3
        pltpu.make_async_copy(rows2.at[q], accsh.at[didx3.at[nchunk - 1, 0]],
                              ssems.at[q]).wait()
        plsc.subcore_barrier()
        off2 = pl.multiple_of(c * _N_PAD + s * _RPW, _RPW)
        pltpu.sync_copy(accsh.at[pl.ds(s * _RPW, _RPW)],
                        out_hbm.at[pl.ds(off2, _RPW)])

    return functools.partial(
        pl.kernel,
        out_type=jax.ShapeDtypeStruct((_NC * _N_PAD, dout), jnp.float32),
        mesh=_mesh,
        compiler_params=_sc_params,
        scratch_types=[
            pltpu.VMEM_SHARED((_N_PAD, dout), jnp.float32),
            pltpu.VMEM((nchunk, 1, chunk), jnp.int32),
            pltpu.VMEM((nchunk, 1, chunk), jnp.int32),
            pltpu.VMEM((3, chunk, dout), jnp.float32),
            pltpu.SemaphoreType.DMA((3,)),
            pltpu.SemaphoreType.DMA((3,)),
        ] + ([pltpu.VMEM_SHARED((_N_PAD, dout), jnp.float32)] if stage else []),
    )(_agg_body)


_agg_chunk = {64: 128, 32: 128, 16: 128, 8: 128}
_agg_kernels = {d: _make_agg(d, _agg_chunk[d]) for d in _agg_chunk}


def _tc0_body(degp_ref, x_ref, w_ref, gh_ref, dinv_ref):
    deg = degp_ref[0, :, 0:1] + degp_ref[1, :, 0:1] + 1.0
    dinv = lax.rsqrt(deg)
    dinv_ref[...] = dinv
    gh_ref[...] = dinv * jnp.dot(x_ref[...], w_ref[...],
                                 preferred_element_type=jnp.float32)


def _tc0(degp, x_pad, w1):
    d1 = _DIMS[1]
    return pl.pallas_call(
        _tc0_body,
        grid=(_NBLK,),
        in_specs=[
            pl.BlockSpec((2, _BLK, 8), lambda i: (0, i, 0)),
            pl.BlockSpec((_BLK, _DIMS[0]), lambda i: (i, 0)),
            pl.BlockSpec((_DIMS[0], d1), lambda i: (0, 0)),
        ],
        out_specs=[
            pl.BlockSpec((_BLK, d1), lambda i: (i, 0)),
            pl.BlockSpec((_BLK, 1), lambda i: (i, 0)),
        ],
        out_shape=[
            jax.ShapeDtypeStruct((_N_PAD, d1), jnp.float32),
            jax.ShapeDtypeStruct((_N_PAD, 1), jnp.float32),
        ],
    )(degp.reshape(2, _N_PAD, 8), x_pad, w1)


def _tcmid_body(a0_ref, a1_ref, gh_ref, dinv_ref, b_ref, w_ref, out_ref):
    i = pl.program_id(0)
    dinv = dinv_ref[...]
    h = jnp.maximum(dinv * (a0_ref[...] + a1_ref[...] + gh_ref[...]) + b_ref[...], 0.0)
    rowid = i * _BLK + lax.broadcasted_iota(jnp.int32, (_BLK, 1), 0)
    h = jnp.where(rowid < _N, h, 0.0)
    out_ref[...] = dinv * jnp.dot(h, w_ref[...], preferred_element_type=jnp.float32)


def _tcmid(accflat, gh, dinv, b2d, wnext, din, dnext):
    return pl.pallas_call(
        _tcmid_body,
        grid=(_NBLK,),
        in_specs=[
            pl.BlockSpec((_BLK, din), lambda i: (i, 0)),
            pl.BlockSpec((_BLK, din), lambda i: (i + _NBLK, 0)),
            pl.BlockSpec((_BLK, din), lambda i: (i, 0)),
            pl.BlockSpec((_BLK, 1), lambda i: (i, 0)),
            pl.BlockSpec((1, din), lambda i: (0, 0)),
            pl.BlockSpec((din, dnext), lambda i: (0, 0)),
        ],
        out_specs=pl.BlockSpec((_BLK, dnext), lambda i: (i, 0)),
        out_shape=jax.ShapeDtypeStruct((_N_PAD, dnext), jnp.float32),
    )(accflat, accflat, gh, dinv, b2d, wnext)


def _tcmid_split_body(a0_ref, a1_ref, gh_ref, dinv_ref, b_ref, wa_ref, wb_ref,
                      outa_ref, outb_ref):
    i = pl.program_id(0)
    dinv = dinv_ref[...]
    h = jnp.maximum(dinv * (a0_ref[...] + a1_ref[...] + gh_ref[...]) + b_ref[...], 0.0)
    rowid = i * _BLK + lax.broadcasted_iota(jnp.int32, (_BLK, 1), 0)
    h = jnp.where(rowid < _N, h, 0.0)
    outa_ref[...] = dinv * jnp.dot(h, wa_ref[...], preferred_element_type=jnp.float32)
    outb_ref[...] = dinv * jnp.dot(h, wb_ref[...], preferred_element_type=jnp.float32)


def _tcmid_split(accflat, gh, dinv, b2d, wa, wb, din, dh):
    return pl.pallas_call(
        _tcmid_split_body,
        grid=(_NBLK,),
        in_specs=[
            pl.BlockSpec((_BLK, din), lambda i: (i, 0)),
            pl.BlockSpec((_BLK, din), lambda i: (i + _NBLK, 0)),
            pl.BlockSpec((_BLK, din), lambda i: (i, 0)),
            pl.BlockSpec((_BLK, 1), lambda i: (i, 0)),
            pl.BlockSpec((1, din), lambda i: (0, 0)),
            pl.BlockSpec((din, dh), lambda i: (0, 0)),
            pl.BlockSpec((din, dh), lambda i: (0, 0)),
        ],
        out_specs=[
            pl.BlockSpec((_BLK, dh), lambda i: (i, 0)),
            pl.BlockSpec((_BLK, dh), lambda i: (i, 0)),
        ],
        out_shape=[
            jax.ShapeDtypeStruct((_N_PAD, dh), jnp.float32),
            jax.ShapeDtypeStruct((_N_PAD, dh), jnp.float32),
        ],
    )(accflat, accflat, gh, dinv, b2d, wa, wb)


def _tc5_body(a0a_ref, a1a_ref, a0b_ref, a1b_ref, gha_ref, ghb_ref, dinv_ref,
              ba_ref, bb_ref, batch_ref, outa_ref, outb_ref,
              sumsa_scr, sumsb_scr, cnt_scr):
    i = pl.program_id(0)

    @pl.when(i == 0)
    def _():
        sumsa_scr[...] = jnp.zeros_like(sumsa_scr)
        sumsb_scr[...] = jnp.zeros_like(sumsb_scr)
        cnt_scr[...] = jnp.zeros_like(cnt_scr)

    dinv = dinv_ref[...]
    rowid = i * _BLK + lax.broadcasted_iota(jnp.int32, (_BLK, 1), 0)
    live = rowid < _N
    ha = jnp.maximum(dinv * (a0a_ref[...] + a1a_ref[...] + gha_ref[...]) + ba_ref[...], 0.0)
    ha = jnp.where(live, ha, 0.0)
    hb = jnp.maximum(dinv * (a0b_ref[...] + a1b_ref[...] + ghb_ref[...]) + bb_ref[...], 0.0)
    hb = jnp.where(live, hb, 0.0)
    gids = lax.broadcasted_iota(jnp.int32, (_BLK, _G), 1)
    mask = (batch_ref[...] == gids).astype(jnp.float32)
    dn = (((0,), (0,)), ((), ()))
    sumsa_scr[...] += lax.dot_general(mask, ha, dn, preferred_element_type=jnp.float32)
    sumsb_scr[...] += lax.dot_general(mask, hb, dn, preferred_element_type=jnp.float32)
    ones = jnp.ones((_BLK, 1), jnp.float32)
    cnt_scr[...] += lax.dot_general(mask, ones, dn, preferred_element_type=jnp.float32)

    @pl.when(i == _NBLK - 1)
    def _():
        c = jnp.maximum(cnt_scr[...], 1.0)
        outa_ref[...] = sumsa_scr[...] / c
        outb_ref[...] = sumsb_scr[...] / c


def _tc5(acca, accb, gha, ghb, dinv, ba2d, bb2d, batch2d):
    dh = 64
    return pl.pallas_call(
        _tc5_body,
        grid=(_NBLK,),
        in_specs=[
            pl.BlockSpec((_BLK, dh), lambda i: (i, 0)),
            pl.BlockSpec((_BLK, dh), lambda i: (i + _NBLK, 0)),
            pl.BlockSpec((_BLK, dh), lambda i: (i, 0)),
            pl.BlockSpec((_BLK, dh), lambda i: (i + _NBLK, 0)),
            pl.BlockSpec((_BLK, dh), lambda i: (i, 0)),
            pl.BlockSpec((_BLK, dh), lambda i: (i, 0)),
            pl.BlockSpec((_BLK, 1), lambda i: (i, 0)),
            pl.BlockSpec((1, dh), lambda i: (0, 0)),
            pl.BlockSpec((1, dh), lambda i: (0, 0)),
            pl.BlockSpec((_BLK, 1), lambda i: (i, 0)),
        ],
        out_specs=[
            pl.BlockSpec((_G, dh), lambda i: (0, 0)),
            pl.BlockSpec((_G, dh), lambda i: (0, 0)),
        ],
        out_shape=[
            jax.ShapeDtypeStruct((_G, dh), jnp.float32),
            jax.ShapeDtypeStruct((_G, dh), jnp.float32),
        ],
        scratch_shapes=[
            pltpu.VMEM((_G, dh), jnp.float32),
            pltpu.VMEM((_G, dh), jnp.float32),
            pltpu.VMEM((_G, 1), jnp.float32),
        ],
    )(acca, acca, accb, accb, gha, ghb, dinv, ba2d, bb2d, batch2d)


def kernel(x, edge_index, batch, W1, b1, W2, b2, W3, b3, W4, b4, W5, b5):
    src = edge_index[0].astype(jnp.int32)
    dst = edge_index[1].astype(jnp.int32)
    pad_e = jnp.full((_E_PAD - _E,), _N, jnp.int32)
    src_f = jnp.concatenate([src, pad_e])
    dst_f = jnp.concatenate([dst, pad_e])
    dst_p = dst_f.reshape(_NW, _NCHUNK, 1, _CHUNK)

    def _eidx(flat, chunk):
        return flat.reshape(_NW, _EPW // chunk, 1, chunk)

    x_pad = jnp.zeros((_N_PAD, _DIMS[0]), jnp.float32).at[:_N].set(x)
    batch2d = jnp.full((_N_PAD, 1), _G, jnp.int32).at[:_N, 0].set(batch.astype(jnp.int32))

    ones8 = jnp.ones((_CHUNK, 8), jnp.float32)
    zr8 = jnp.zeros((_RPW, 8), jnp.float32)

    degp = _deg_kernel(dst_p, ones8, zr8)
    gh, dinv = _tc0(degp, x_pad, W1)

    params = [(W2, b1), (W3, b2), (W4, b3)]
    for l in range(1, 4):
        din, dnext = _DIMS[l], _DIMS[l + 1]
        ch = _agg_chunk[din]
        zr = jnp.zeros((_RPW, din), jnp.float32)
        accflat = _agg_kernels[din](gh, _eidx(src_f, ch), _eidx(dst_f, ch), zr)
        wnext, b = params[l - 1]
        gh = _tcmid(accflat, gh, dinv, b.reshape(1, din), wnext, din, dnext)

    din = _DIMS[4]
    ch = _agg_chunk[din]
    zr = jnp.zeros((_RPW, din), jnp.float32)
    accflat = _agg_kernels[din](gh, _eidx(src_f, ch), _eidx(dst_f, ch), zr)
    gha, ghb = _tcmid_split(accflat, gh, dinv, b4.reshape(1, din),
                            W5[:, :64], W5[:, 64:], din, 64)

    ch = _agg_chunk[64]
    zr = jnp.zeros((_RPW, 64), jnp.float32)
    acca = _agg_kernels[64](gha, _eidx(src_f, ch), _eidx(dst_f, ch), zr)
    accb = _agg_kernels[64](ghb, _eidx(src_f, ch), _eidx(dst_f, ch), zr)
    outa, outb = _tc5(acca, accb, gha, ghb, dinv,
                      b5[:64].reshape(1, 64), b5[64:].reshape(1, 64), batch2d)
    return jnp.concatenate([outa, outb], axis=1)


# speedup vs baseline: 28.4188x; 1.0331x over previous
import functools
import jax
import jax.numpy as jnp
from jax import lax
from jax.experimental import pallas as pl
from jax.experimental.pallas import tpu as pltpu
from jax.experimental.pallas import tpu_sc as plsc

_N = 10000
_E = 320000
_G = 16
_DIMS = [128, 64, 32, 16, 8, 128]

_NC = 2
_NS = 16
_NW = _NC * _NS

_CHUNK = 128
_EPW = 10240
_E_PAD = _NW * _EPW
_NCHUNK = _EPW // _CHUNK
_N_PAD = 10240
_RPW = _N_PAD // _NS
_BLK = 512
_NBLK = _N_PAD // _BLK

_mesh = plsc.VectorSubcoreMesh(core_axis_name="c", subcore_axis_name="s")


def _deg_body(dst_hbm, ones_hbm, zr_hbm, out_hbm, accsh, didx3, onesv, sem):
    c = lax.axis_index("c")
    s = lax.axis_index("s")
    w = s * _NC + c
    pltpu.sync_copy(zr_hbm, accsh.at[pl.ds(s * _RPW, _RPW)])
    pltpu.sync_copy(ones_hbm, onesv)
    pltpu.sync_copy(dst_hbm.at[w], didx3)
    plsc.subcore_barrier()

    def body(i, _):
        pltpu.sync_copy(onesv, accsh.at[didx3.at[i, 0]], add=True)
        return _

    lax.fori_loop(0, _NCHUNK, body, None)
    plsc.subcore_barrier()
    off2 = pl.multiple_of(c * _N_PAD + s * _RPW, _RPW)
    pltpu.sync_copy(accsh.at[pl.ds(s * _RPW, _RPW)], out_hbm.at[pl.ds(off2, _RPW)])


_sc_params = pltpu.CompilerParams(use_tc_tiling_on_sc=False)

_deg_kernel = functools.partial(
    pl.kernel,
    out_type=jax.ShapeDtypeStruct((_NC * _N_PAD, 8), jnp.float32),
    mesh=_mesh,
    compiler_params=_sc_params,
    scratch_types=[
        pltpu.VMEM_SHARED((_N_PAD, 8), jnp.float32),
        pltpu.VMEM((_NCHUNK, 1, _CHUNK), jnp.int32),
        pltpu.VMEM((_CHUNK, 8), jnp.float32),
        pltpu.SemaphoreType.DMA,
    ],
)(_deg_body)


def _make_agg(dout, chunk):
    nchunk = _EPW // chunk
    stage = dout <= 64

    def _agg_body(gh_hbm, src_hbm, dst_hbm, zr_hbm, out_hbm, accsh, sidx3,
                  didx3, rows2, gsems, ssems, *maybe_ghs):
        c = lax.axis_index("c")
        s = lax.axis_index("s")
        w = s * _NC + c
        d0 = pltpu.async_copy(zr_hbm, accsh.at[pl.ds(s * _RPW, _RPW)],
                              gsems.at[0])
        d1 = pltpu.async_copy(src_hbm.at[w], sidx3, gsems.at[1])
        d2 = pltpu.async_copy(dst_hbm.at[w], didx3, gsems.at[2])
        if stage:
            ghs = maybe_ghs[0]
            d3 = pltpu.async_copy(gh_hbm.at[pl.ds(s * _RPW, _RPW)],
                                  ghs.at[pl.ds(s * _RPW, _RPW)], ssems.at[0])
            d3.wait()
            gh_src = ghs
        else:
            gh_src = gh_hbm
        d0.wait()
        d1.wait()
        d2.wait()
        plsc.subcore_barrier()

        pltpu.async_copy(gh_src.at[sidx3.at[0, 0]], rows2.at[0], gsems.at[0])
        pltpu.async_copy(gh_src.at[sidx3.at[1, 0]], rows2.at[1], gsems.at[1])

        def body(j, _):
            p = lax.rem(j, 3)

            @pl.when(j >= 1)
            def _():
                q = lax.rem(j - 1, 3)
                pltpu.make_async_copy(rows2.at[q],
                                      accsh.at[didx3.at[j - 1, 0]],
                                      ssems.at[q]).wait()

            @pl.when(j + 2 < nchunk)
            def _():
                q = lax.rem(j + 2, 3)
                pltpu.async_copy(gh_src.at[sidx3.at[j + 2, 0]], rows2.at[q],
                                 gsems.at[q])

            pltpu.make_async_copy(gh_src.at[sidx3.at[j, 0]], rows2.at[p],
                                  gsems.at[p]).wait()
            pltpu.async_copy(rows2.at[p], accsh.at[didx3.at[j, 0]],
                             ssems.at[p], add=True)
            return _

        lax.fori_loop(0, nchunk, body, None)
        q = (nchunk - 1) % 3
        pltpu.make_async_copy(rows2.at[q], accsh.at[didx3.at[nchunk - 1, 0]],
                              ssems.at[q]).wait()
        plsc.subcore_barrier()
        off2 = pl.multiple_of(c * _N_PAD + s * _RPW, _RPW)
        pltpu.sync_copy(accsh.at[pl.ds(s * _RPW, _RPW)],
                        out_hbm.at[pl.ds(off2, _RPW)])

    return functools.partial(
        pl.kernel,
        out_type=jax.ShapeDtypeStruct((_NC * _N_PAD, dout), jnp.float32),
        mesh=_mesh,
        compiler_params=_sc_params,
        scratch_types=[
            pltpu.VMEM_SHARED((_N_PAD, dout), jnp.float32),
            pltpu.VMEM((nchunk, 1, chunk), jnp.int32),
            pltpu.VMEM((nchunk, 1, chunk), jnp.int32),
            pltpu.VMEM((3, chunk, dout), jnp.float32),
            pltpu.SemaphoreType.DMA((3,)),
            pltpu.SemaphoreType.DMA((3,)),
        ] + ([pltpu.VMEM_SHARED((_N_PAD, dout), jnp.float32)] if stage else []),
    )(_agg_body)


_agg_chunk = {64: 128, 32: 128, 16: 128, 8: 128}
_agg_kernels = {d: _make_agg(d, _agg_chunk[d]) for d in _agg_chunk}


def _make_agg_fused(dh, chunk):
    nchunk = _EPW // chunk

    def _body(gha_hbm, ghb_hbm, src_hbm, dst_hbm, zr_hbm, out_hbm, accsh,
              sidx3, didx3, rows2, gsems, ssems, ghs):
        c = lax.axis_index("c")
        s = lax.axis_index("s")
        d0 = pltpu.async_copy(zr_hbm, accsh.at[pl.ds(s * _RPW, _RPW)],
                              gsems.at[0])

        @pl.when(c == 0)
        def _():
            pltpu.sync_copy(gha_hbm.at[pl.ds(s * _RPW, _RPW)],
                            ghs.at[pl.ds(s * _RPW, _RPW)])

        @pl.when(c == 1)
        def _():
            pltpu.sync_copy(ghb_hbm.at[pl.ds(s * _RPW, _RPW)],
                            ghs.at[pl.ds(s * _RPW, _RPW)])

        d0.wait()
        plsc.subcore_barrier()

        for ph in range(2):
            w = s * 2 + ph
            pltpu.sync_copy(src_hbm.at[w], sidx3)
            pltpu.sync_copy(dst_hbm.at[w], didx3)

            pltpu.async_copy(ghs.at[sidx3.at[0, 0]], rows2.at[0], gsems.at[0])
            pltpu.async_copy(ghs.at[sidx3.at[1, 0]], rows2.at[1], gsems.at[1])

            def body(j, _):
                p = lax.rem(j, 3)

                @pl.when(j >= 1)
                def _():
                    q = lax.rem(j - 1, 3)
                    pltpu.make_async_copy(rows2.at[q],
                                          accsh.at[didx3.at[j - 1, 0]],
                                          ssems.at[q]).wait()

                @pl.when(j + 2 < nchunk)
                def _():
                    q = lax.rem(j + 2, 3)
                    pltpu.async_copy(ghs.at[sidx3.at[j + 2, 0]], rows2.at[q],
                                     gsems.at[q])

                pltpu.make_async_copy(ghs.at[sidx3.at[j, 0]], rows2.at[p],
                                      gsems.at[p]).wait()
                pltpu.async_copy(rows2.at[p], accsh.at[didx3.at[j, 0]],
                                 ssems.at[p], add=True)
                return _

            lax.fori_loop(0, nchunk, body, None)
            q = (nchunk - 1) % 3
            pltpu.make_async_copy(rows2.at[q],
                                  accsh.at[didx3.at[nchunk - 1, 0]],
                                  ssems.at[q]).wait()

        plsc.subcore_barrier()
        off2 = pl.multiple_of(c * _N_PAD + s * _RPW, _RPW)
        pltpu.sync_copy(accsh.at[pl.ds(s * _RPW, _RPW)],
                        out_hbm.at[pl.ds(off2, _RPW)])

    return functools.partial(
        pl.kernel,
        out_type=jax.ShapeDtypeStruct((_NC * _N_PAD, dh), jnp.float32),
        mesh=_mesh,
        compiler_params=_sc_params,
        scratch_types=[
            pltpu.VMEM_SHARED((_N_PAD, dh), jnp.float32),
            pltpu.VMEM((nchunk, 1, chunk), jnp.int32),
            pltpu.VMEM((nchunk, 1, chunk), jnp.int32),
            pltpu.VMEM((3, chunk, dh), jnp.float32),
            pltpu.SemaphoreType.DMA((3,)),
            pltpu.SemaphoreType.DMA((3,)),
            pltpu.VMEM_SHARED((_N_PAD, dh), jnp.float32),
        ],
    )(_body)


_agg5_kernel = _make_agg_fused(64, 128)


def _tc0_body(degp_ref, x_ref, w_ref, gh_ref, dinv_ref):
    deg = degp_ref[0, :, 0:1] + degp_ref[1, :, 0:1] + 1.0
    dinv = lax.rsqrt(deg)
    dinv_ref[...] = dinv
    gh_ref[...] = dinv * jnp.dot(x_ref[...], w_ref[...],
                                 preferred_element_type=jnp.float32)


def _tc0(degp, x_pad, w1):
    d1 = _DIMS[1]
    return pl.pallas_call(
        _tc0_body,
        grid=(_NBLK,),
        in_specs=[
            pl.BlockSpec((2, _BLK, 8), lambda i: (0, i, 0)),
            pl.BlockSpec((_BLK, _DIMS[0]), lambda i: (i, 0)),
            pl.BlockSpec((_DIMS[0], d1), lambda i: (0, 0)),
        ],
        out_specs=[
            pl.BlockSpec((_BLK, d1), lambda i: (i, 0)),
            pl.BlockSpec((_BLK, 1), lambda i: (i, 0)),
        ],
        out_shape=[
            jax.ShapeDtypeStruct((_N_PAD, d1), jnp.float32),
            jax.ShapeDtypeStruct((_N_PAD, 1), jnp.float32),
        ],
    )(degp.reshape(2, _N_PAD, 8), x_pad, w1)


def _tcmid_body(a0_ref, a1_ref, gh_ref, dinv_ref, b_ref, w_ref, out_ref):
    i = pl.program_id(0)
    dinv = dinv_ref[...]
    h = jnp.maximum(dinv * (a0_ref[...] + a1_ref[...] + gh_ref[...]) + b_ref[...], 0.0)
    rowid = i * _BLK + lax.broadcasted_iota(jnp.int32, (_BLK, 1), 0)
    h = jnp.where(rowid < _N, h, 0.0)
    out_ref[...] = dinv * jnp.dot(h, w_ref[...], preferred_element_type=jnp.float32)


def _tcmid(accflat, gh, dinv, b2d, wnext, din, dnext):
    return pl.pallas_call(
        _tcmid_body,
        grid=(_NBLK,),
        in_specs=[
            pl.BlockSpec((_BLK, din), lambda i: (i, 0)),
            pl.BlockSpec((_BLK, din), lambda i: (i + _NBLK, 0)),
            pl.BlockSpec((_BLK, din), lambda i: (i, 0)),
            pl.BlockSpec((_BLK, 1), lambda i: (i, 0)),
            pl.BlockSpec((1, din), lambda i: (0, 0)),
            pl.BlockSpec((din, dnext), lambda i: (0, 0)),
        ],
        out_specs=pl.BlockSpec((_BLK, dnext), lambda i: (i, 0)),
        out_shape=jax.ShapeDtypeStruct((_N_PAD, dnext), jnp.float32),
    )(accflat, accflat, gh, dinv, b2d, wnext)


def _tcmid_split_body(a0_ref, a1_ref, gh_ref, dinv_ref, b_ref, wa_ref, wb_ref,
                      outa_ref, outb_ref):
    i = pl.program_id(0)
    dinv = dinv_ref[...]
    h = jnp.maximum(dinv * (a0_ref[...] + a1_ref[...] + gh_ref[...]) + b_ref[...], 0.0)
    rowid = i * _BLK + lax.broadcasted_iota(jnp.int32, (_BLK, 1), 0)
    h = jnp.where(rowid < _N, h, 0.0)
    outa_ref[...] = dinv * jnp.dot(h, wa_ref[...], preferred_element_type=jnp.float32)
    outb_ref[...] = dinv * jnp.dot(h, wb_ref[...], preferred_element_type=jnp.float32)


def _tcmid_split(accflat, gh, dinv, b2d, wa, wb, din, dh):
    return pl.pallas_call(
        _tcmid_split_body,
        grid=(_NBLK,),
        in_specs=[
            pl.BlockSpec((_BLK, din), lambda i: (i, 0)),
            pl.BlockSpec((_BLK, din), lambda i: (i + _NBLK, 0)),
            pl.BlockSpec((_BLK, din), lambda i: (i, 0)),
            pl.BlockSpec((_BLK, 1), lambda i: (i, 0)),
            pl.BlockSpec((1, din), lambda i: (0, 0)),
            pl.BlockSpec((din, dh), lambda i: (0, 0)),
            pl.BlockSpec((din, dh), lambda i: (0, 0)),
        ],
        out_specs=[
            pl.BlockSpec((_BLK, dh), lambda i: (i, 0)),
            pl.BlockSpec((_BLK, dh), lambda i: (i, 0)),
        ],
        out_shape=[
            jax.ShapeDtypeStruct((_N_PAD, dh), jnp.float32),
            jax.ShapeDtypeStruct((_N_PAD, dh), jnp.float32),
        ],
    )(accflat, accflat, gh, dinv, b2d, wa, wb)


def _tc5_body(aa_ref, ab_ref, gha_ref, ghb_ref, dinv_ref,
              ba_ref, bb_ref, batch_ref, outa_ref, outb_ref,
              sumsa_scr, sumsb_scr, cnt_scr):
    i = pl.program_id(0)

    @pl.when(i == 0)
    def _():
        sumsa_scr[...] = jnp.zeros_like(sumsa_scr)
        sumsb_scr[...] = jnp.zeros_like(sumsb_scr)
        cnt_scr[...] = jnp.zeros_like(cnt_scr)

    dinv = dinv_ref[...]
    rowid = i * _BLK + lax.broadcasted_iota(jnp.int32, (_BLK, 1), 0)
    live = rowid < _N
    ha = jnp.maximum(dinv * (aa_ref[...] + gha_ref[...]) + ba_ref[...], 0.0)
    ha = jnp.where(live, ha, 0.0)
    hb = jnp.maximum(dinv * (ab_ref[...] + ghb_ref[...]) + bb_ref[...], 0.0)
    hb = jnp.where(live, hb, 0.0)
    gids = lax.broadcasted_iota(jnp.int32, (_BLK, _G), 1)
    mask = (batch_ref[...] == gids).astype(jnp.float32)
    dn = (((0,), (0,)), ((), ()))
    sumsa_scr[...] += lax.dot_general(mask, ha, dn, preferred_element_type=jnp.float32)
    sumsb_scr[...] += lax.dot_general(mask, hb, dn, preferred_element_type=jnp.float32)
    ones = jnp.ones((_BLK, 1), jnp.float32)
    cnt_scr[...] += lax.dot_general(mask, ones, dn, preferred_element_type=jnp.float32)

    @pl.when(i == _NBLK - 1)
    def _():
        c = jnp.maximum(cnt_scr[...], 1.0)
        outa_ref[...] = sumsa_scr[...] / c
        outb_ref[...] = sumsb_scr[...] / c


def _tc5(accab, gha, ghb, dinv, ba2d, bb2d, batch2d):
    dh = 64
    return pl.pallas_call(
        _tc5_body,
        grid=(_NBLK,),
        in_specs=[
            pl.BlockSpec((_BLK, dh), lambda i: (i, 0)),
            pl.BlockSpec((_BLK, dh), lambda i: (i + _NBLK, 0)),
            pl.BlockSpec((_BLK, dh), lambda i: (i, 0)),
            pl.BlockSpec((_BLK, dh), lambda i: (i, 0)),
            pl.BlockSpec((_BLK, 1), lambda i: (i, 0)),
            pl.BlockSpec((1, dh), lambda i: (0, 0)),
            pl.BlockSpec((1, dh), lambda i: (0, 0)),
            pl.BlockSpec((_BLK, 1), lambda i: (i, 0)),
        ],
        out_specs=[
            pl.BlockSpec((_G, dh), lambda i: (0, 0)),
            pl.BlockSpec((_G, dh), lambda i: (0, 0)),
        ],
        out_shape=[
            jax.ShapeDtypeStruct((_G, dh), jnp.float32),
            jax.ShapeDtypeStruct((_G, dh), jnp.float32),
        ],
        scratch_shapes=[
            pltpu.VMEM((_G, dh), jnp.float32),
            pltpu.VMEM((_G, dh), jnp.float32),
            pltpu.VMEM((_G, 1), jnp.float32),
        ],
    )(accab, accab, gha, ghb, dinv, ba2d, bb2d, batch2d)


def kernel(x, edge_index, batch, W1, b1, W2, b2, W3, b3, W4, b4, W5, b5):
    src = edge_index[0].astype(jnp.int32)
    dst = edge_index[1].astype(jnp.int32)
    pad_e = jnp.full((_E_PAD - _E,), _N, jnp.int32)
    src_f = jnp.concatenate([src, pad_e])
    dst_f = jnp.concatenate([dst, pad_e])
    dst_p = dst_f.reshape(_NW, _NCHUNK, 1, _CHUNK)

    def _eidx(flat, chunk):
        return flat.reshape(_NW, _EPW // chunk, 1, chunk)

    x_pad = jnp.zeros((_N_PAD, _DIMS[0]), jnp.float32).at[:_N].set(x)
    batch2d = jnp.full((_N_PAD, 1), _G, jnp.int32).at[:_N, 0].set(batch.astype(jnp.int32))

    ones8 = jnp.ones((_CHUNK, 8), jnp.float32)
    zr8 = jnp.zeros((_RPW, 8), jnp.float32)

    degp = _deg_kernel(dst_p, ones8, zr8)
    gh, dinv = _tc0(degp, x_pad, W1)

    params = [(W2, b1), (W3, b2), (W4, b3)]
    for l in range(1, 4):
        din, dnext = _DIMS[l], _DIMS[l + 1]
        ch = _agg_chunk[din]
        zr = jnp.zeros((_RPW, din), jnp.float32)
        accflat = _agg_kernels[din](gh, _eidx(src_f, ch), _eidx(dst_f, ch), zr)
        wnext, b = params[l - 1]
        gh = _tcmid(accflat, gh, dinv, b.reshape(1, din), wnext, din, dnext)

    din = _DIMS[4]
    ch = _agg_chunk[din]
    zr = jnp.zeros((_RPW, din), jnp.float32)
    accflat = _agg_kernels[din](gh, _eidx(src_f, ch), _eidx(dst_f, ch), zr)
    gha, ghb = _tcmid_split(accflat, gh, dinv, b4.reshape(1, din),
                            W5[:, :64], W5[:, 64:], din, 64)

    ch = _agg_chunk[64]
    zr = jnp.zeros((_RPW, 64), jnp.float32)
    accab = _agg5_kernel(gha, ghb, _eidx(src_f, ch), _eidx(dst_f, ch), zr)
    outa, outb = _tc5(accab, gha, ghb, dinv,
                      b5[:64].reshape(1, 64), b5[64:].reshape(1, 64), batch2d)
    return jnp.concatenate([outa, outb], axis=1)


# speedup vs baseline: 37.8271x; 1.3311x over previous
import functools
import jax
import jax.numpy as jnp
from jax import lax
from jax.experimental import pallas as pl
from jax.experimental.pallas import tpu as pltpu
from jax.experimental.pallas import tpu_sc as plsc

_N = 10000
_E = 320000
_G = 16
_DIMS = [128, 64, 32, 16, 8, 128]

_NC = 2
_NS = 16
_NW = _NC * _NS

_CHUNK = 128
_EPW = 10240
_E_PAD = _NW * _EPW
_NCHUNK = _EPW // _CHUNK
_N_PAD = 10240
_RPW = _N_PAD // _NS
_BLK = 512
_NBLK = _N_PAD // _BLK

_mesh = plsc.VectorSubcoreMesh(core_axis_name="c", subcore_axis_name="s")


def _deg_body(dst_hbm, ones_hbm, zr_hbm, out_hbm, accsh, didx3, onesv, sem):
    c = lax.axis_index("c")
    s = lax.axis_index("s")
    w = s * _NC + c
    pltpu.sync_copy(zr_hbm, accsh.at[pl.ds(s * _RPW, _RPW)])
    pltpu.sync_copy(ones_hbm, onesv)
    pltpu.sync_copy(dst_hbm.at[w], didx3)
    plsc.subcore_barrier()

    def body(i, _):
        pltpu.sync_copy(onesv, accsh.at[didx3.at[i, 0]], add=True)
        return _

    lax.fori_loop(0, _NCHUNK, body, None)
    plsc.subcore_barrier()
    off2 = pl.multiple_of(c * _N_PAD + s * _RPW, _RPW)
    pltpu.sync_copy(accsh.at[pl.ds(s * _RPW, _RPW)], out_hbm.at[pl.ds(off2, _RPW)])


_sc_params = pltpu.CompilerParams(use_tc_tiling_on_sc=False)

_deg_kernel = functools.partial(
    pl.kernel,
    out_type=jax.ShapeDtypeStruct((_NC * _N_PAD, 8), jnp.float32),
    mesh=_mesh,
    compiler_params=_sc_params,
    scratch_types=[
        pltpu.VMEM_SHARED((_N_PAD, 8), jnp.float32),
        pltpu.VMEM((_NCHUNK, 1, _CHUNK), jnp.int32),
        pltpu.VMEM((_CHUNK, 8), jnp.float32),
        pltpu.SemaphoreType.DMA,
    ],
)(_deg_body)


def _make_agg(dout, chunk):
    nchunk = _EPW // chunk
    stage = dout <= 64

    def _agg_body(gh_hbm, src_hbm, dst_hbm, zr_hbm, out_hbm, accsh, sidx3,
                  didx3, rows2, gsems, ssems, *maybe_ghs):
        c = lax.axis_index("c")
        s = lax.axis_index("s")
        w = s * _NC + c
        d0 = pltpu.async_copy(zr_hbm, accsh.at[pl.ds(s * _RPW, _RPW)],
                              gsems.at[0])
        d1 = pltpu.async_copy(src_hbm.at[w], sidx3, gsems.at[1])
        d2 = pltpu.async_copy(dst_hbm.at[w], didx3, gsems.at[2])
        if stage:
            ghs = maybe_ghs[0]
            d3 = pltpu.async_copy(gh_hbm.at[pl.ds(s * _RPW, _RPW)],
                                  ghs.at[pl.ds(s * _RPW, _RPW)], ssems.at[0])
            d3.wait()
            gh_src = ghs
        else:
            gh_src = gh_hbm
        d0.wait()
        d1.wait()
        d2.wait()
        plsc.subcore_barrier()

        pltpu.async_copy(gh_src.at[sidx3.at[0, 0]], rows2.at[0], gsems.at[0])
        pltpu.async_copy(gh_src.at[sidx3.at[1, 0]], rows2.at[1], gsems.at[1])

        def body(j, _):
            p = lax.rem(j, 3)

            @pl.when(j >= 1)
            def _():
                q = lax.rem(j - 1, 3)
                pltpu.make_async_copy(rows2.at[q],
                                      accsh.at[didx3.at[j - 1, 0]],
                                      ssems.at[q]).wait()

            @pl.when(j + 2 < nchunk)
            def _():
                q = lax.rem(j + 2, 3)
                pltpu.async_copy(gh_src.at[sidx3.at[j + 2, 0]], rows2.at[q],
                                 gsems.at[q])

            pltpu.make_async_copy(gh_src.at[sidx3.at[j, 0]], rows2.at[p],
                                  gsems.at[p]).wait()
            pltpu.async_copy(rows2.at[p], accsh.at[didx3.at[j, 0]],
                             ssems.at[p], add=True)
            return _

        lax.fori_loop(0, nchunk, body, None)
        q = (nchunk - 1) % 3
        pltpu.make_async_copy(rows2.at[q], accsh.at[didx3.at[nchunk - 1, 0]],
                              ssems.at[q]).wait()
        plsc.subcore_barrier()
        off2 = pl.multiple_of(c * _N_PAD + s * _RPW, _RPW)
        pltpu.sync_copy(accsh.at[pl.ds(s * _RPW, _RPW)],
                        out_hbm.at[pl.ds(off2, _RPW)])

    return functools.partial(
        pl.kernel,
        out_type=jax.ShapeDtypeStruct((_NC * _N_PAD, dout), jnp.float32),
        mesh=_mesh,
        compiler_params=_sc_params,
        scratch_types=[
            pltpu.VMEM_SHARED((_N_PAD, dout), jnp.float32),
            pltpu.VMEM((nchunk, 1, chunk), jnp.int32),
            pltpu.VMEM((nchunk, 1, chunk), jnp.int32),
            pltpu.VMEM((3, chunk, dout), jnp.float32),
            pltpu.SemaphoreType.DMA((3,)),
            pltpu.SemaphoreType.DMA((3,)),
        ] + ([pltpu.VMEM_SHARED((_N_PAD, dout), jnp.float32)] if stage else []),
    )(_agg_body)


_agg_chunk = {64: 128, 32: 128, 16: 128, 8: 128}
_agg_kernels = {d: _make_agg(d, _agg_chunk[d]) for d in _agg_chunk}


def _tc0_body(degp_ref, x_ref, w_ref, gh_ref, dinv_ref):
    deg = degp_ref[0, :, 0:1] + degp_ref[1, :, 0:1] + 1.0
    dinv = lax.rsqrt(deg)
    dinv_ref[...] = dinv
    gh_ref[...] = dinv * jnp.dot(x_ref[...], w_ref[...],
                                 preferred_element_type=jnp.float32)


def _tc0(degp, x_pad, w1):
    d1 = _DIMS[1]
    return pl.pallas_call(
        _tc0_body,
        grid=(_NBLK,),
        in_specs=[
            pl.BlockSpec((2, _BLK, 8), lambda i: (0, i, 0)),
            pl.BlockSpec((_BLK, _DIMS[0]), lambda i: (i, 0)),
            pl.BlockSpec((_DIMS[0], d1), lambda i: (0, 0)),
        ],
        out_specs=[
            pl.BlockSpec((_BLK, d1), lambda i: (i, 0)),
            pl.BlockSpec((_BLK, 1), lambda i: (i, 0)),
        ],
        out_shape=[
            jax.ShapeDtypeStruct((_N_PAD, d1), jnp.float32),
            jax.ShapeDtypeStruct((_N_PAD, 1), jnp.float32),
        ],
    )(degp.reshape(2, _N_PAD, 8), x_pad, w1)


def _tcmid_body(a0_ref, a1_ref, gh_ref, dinv_ref, b_ref, w_ref, out_ref):
    i = pl.program_id(0)
    dinv = dinv_ref[...]
    h = jnp.maximum(dinv * (a0_ref[...] + a1_ref[...] + gh_ref[...]) + b_ref[...], 0.0)
    rowid = i * _BLK + lax.broadcasted_iota(jnp.int32, (_BLK, 1), 0)
    h = jnp.where(rowid < _N, h, 0.0)
    out_ref[...] = dinv * jnp.dot(h, w_ref[...], preferred_element_type=jnp.float32)


def _tcmid(accflat, gh, dinv, b2d, wnext, din, dnext):
    return pl.pallas_call(
        _tcmid_body,
        grid=(_NBLK,),
        in_specs=[
            pl.BlockSpec((_BLK, din), lambda i: (i, 0)),
            pl.BlockSpec((_BLK, din), lambda i: (i + _NBLK, 0)),
            pl.BlockSpec((_BLK, din), lambda i: (i, 0)),
            pl.BlockSpec((_BLK, 1), lambda i: (i, 0)),
            pl.BlockSpec((1, din), lambda i: (0, 0)),
            pl.BlockSpec((din, dnext), lambda i: (0, 0)),
        ],
        out_specs=pl.BlockSpec((_BLK, dnext), lambda i: (i, 0)),
        out_shape=jax.ShapeDtypeStruct((_N_PAD, dnext), jnp.float32),
    )(accflat, accflat, gh, dinv, b2d, wnext)


def _tcmid_nomm_body(a0_ref, a1_ref, gh_ref, dinv_ref, b_ref, out_ref):
    i = pl.program_id(0)
    dinv = dinv_ref[...]
    h = jnp.maximum(dinv * (a0_ref[...] + a1_ref[...] + gh_ref[...]) + b_ref[...], 0.0)
    rowid = i * _BLK + lax.broadcasted_iota(jnp.int32, (_BLK, 1), 0)
    h = jnp.where(rowid < _N, h, 0.0)
    out_ref[...] = dinv * h


def _tcmid_nomm(accflat, gh, dinv, b2d, din):
    return pl.pallas_call(
        _tcmid_nomm_body,
        grid=(_NBLK,),
        in_specs=[
            pl.BlockSpec((_BLK, din), lambda i: (i, 0)),
            pl.BlockSpec((_BLK, din), lambda i: (i + _NBLK, 0)),
            pl.BlockSpec((_BLK, din), lambda i: (i, 0)),
            pl.BlockSpec((_BLK, 1), lambda i: (i, 0)),
            pl.BlockSpec((1, din), lambda i: (0, 0)),
        ],
        out_specs=pl.BlockSpec((_BLK, din), lambda i: (i, 0)),
        out_shape=jax.ShapeDtypeStruct((_N_PAD, din), jnp.float32),
    )(accflat, accflat, gh, dinv, b2d)


def _tc5_body(a0_ref, a1_ref, ghp_ref, dinv_ref, w_ref, b_ref, batch_ref,
              out_ref, sums_scr, cnt_scr):
    i = pl.program_id(0)

    @pl.when(i == 0)
    def _():
        sums_scr[...] = jnp.zeros_like(sums_scr)
        cnt_scr[...] = jnp.zeros_like(cnt_scr)

    dinv = dinv_ref[...]
    t = dinv * (a0_ref[...] + a1_ref[...] + ghp_ref[...])
    h = jnp.maximum(jnp.dot(t, w_ref[...], preferred_element_type=jnp.float32)
                    + b_ref[...], 0.0)
    rowid = i * _BLK + lax.broadcasted_iota(jnp.int32, (_BLK, 1), 0)
    h = jnp.where(rowid < _N, h, 0.0)
    gids = lax.broadcasted_iota(jnp.int32, (_BLK, _G), 1)
    mask = (batch_ref[...] == gids).astype(jnp.float32)
    dn = (((0,), (0,)), ((), ()))
    sums_scr[...] += lax.dot_general(mask, h, dn, preferred_element_type=jnp.float32)
    ones = jnp.ones((_BLK, 1), jnp.float32)
    cnt_scr[...] += lax.dot_general(mask, ones, dn, preferred_element_type=jnp.float32)

    @pl.when(i == _NBLK - 1)
    def _():
        out_ref[...] = sums_scr[...] / jnp.maximum(cnt_scr[...], 1.0)


def _tc5(accflat, ghp, dinv, w5, b2d, batch2d):
    din, d5 = _DIMS[4], _DIMS[5]
    return pl.pallas_call(
        _tc5_body,
        grid=(_NBLK,),
        in_specs=[
            pl.BlockSpec((_BLK, din), lambda i: (i, 0)),
            pl.BlockSpec((_BLK, din), lambda i: (i + _NBLK, 0)),
            pl.BlockSpec((_BLK, din), lambda i: (i, 0)),
            pl.BlockSpec((_BLK, 1), lambda i: (i, 0)),
            pl.BlockSpec((din, d5), lambda i: (0, 0)),
            pl.BlockSpec((1, d5), lambda i: (0, 0)),
            pl.BlockSpec((_BLK, 1), lambda i: (i, 0)),
        ],
        out_specs=pl.BlockSpec((_G, d5), lambda i: (0, 0)),
        out_shape=jax.ShapeDtypeStruct((_G, d5), jnp.float32),
        scratch_shapes=[
            pltpu.VMEM((_G, d5), jnp.float32),
            pltpu.VMEM((_G, 1), jnp.float32),
        ],
    )(accflat, accflat, ghp, dinv, w5, b2d, batch2d)


def kernel(x, edge_index, batch, W1, b1, W2, b2, W3, b3, W4, b4, W5, b5):
    src = edge_index[0].astype(jnp.int32)
    dst = edge_index[1].astype(jnp.int32)
    pad_e = jnp.full((_E_PAD - _E,), _N, jnp.int32)
    src_f = jnp.concatenate([src, pad_e])
    dst_f = jnp.concatenate([dst, pad_e])
    dst_p = dst_f.reshape(_NW, _NCHUNK, 1, _CHUNK)

    def _eidx(flat, chunk):
        return flat.reshape(_NW, _EPW // chunk, 1, chunk)

    x_pad = jnp.zeros((_N_PAD, _DIMS[0]), jnp.float32).at[:_N].set(x)
    batch2d = jnp.full((_N_PAD, 1), _G, jnp.int32).at[:_N, 0].set(batch.astype(jnp.int32))

    ones8 = jnp.ones((_CHUNK, 8), jnp.float32)
    zr8 = jnp.zeros((_RPW, 8), jnp.float32)

    degp = _deg_kernel(dst_p, ones8, zr8)
    gh, dinv = _tc0(degp, x_pad, W1)

    params = [(W2, b1), (W3, b2), (W4, b3)]
    for l in range(1, 4):
        din, dnext = _DIMS[l], _DIMS[l + 1]
        ch = _agg_chunk[din]
        zr = jnp.zeros((_RPW, din), jnp.float32)
        accflat = _agg_kernels[din](gh, _eidx(src_f, ch), _eidx(dst_f, ch), zr)
        wnext, b = params[l - 1]
        gh = _tcmid(accflat, gh, dinv, b.reshape(1, din), wnext, din, dnext)

    din = _DIMS[4]
    ch = _agg_chunk[din]
    zr = jnp.zeros((_RPW, din), jnp.float32)
    accflat = _agg_kernels[din](gh, _eidx(src_f, ch), _eidx(dst_f, ch), zr)
    ghp = _tcmid_nomm(accflat, gh, dinv, b4.reshape(1, din), din)

    accflat = _agg_kernels[din](ghp, _eidx(src_f, ch), _eidx(dst_f, ch), zr)
    return _tc5(accflat, ghp, dinv, W5, b5.reshape(1, _DIMS[5]), batch2d)


# speedup vs baseline: 38.2049x; 1.0100x over previous
import functools
import jax
import jax.numpy as jnp
from jax import lax
from jax.experimental import pallas as pl
from jax.experimental.pallas import tpu as pltpu
from jax.experimental.pallas import tpu_sc as plsc

_N = 10000
_E = 320000
_G = 16
_DIMS = [128, 64, 32, 16, 8, 128]

_NC = 2
_NS = 16
_NW = _NC * _NS

_CHUNK = 128
_EPW = 10240
_E_PAD = _NW * _EPW
_NCHUNK = _EPW // _CHUNK
_N_PAD = 10240
_RPW = _N_PAD // _NS
_BLK = 512
_NBLK = _N_PAD // _BLK

_mesh = plsc.VectorSubcoreMesh(core_axis_name="c", subcore_axis_name="s")


def _deg_body(dst_hbm, ones_hbm, zr_hbm, out_hbm, accsh, didx3, onesv, sem):
    c = lax.axis_index("c")
    s = lax.axis_index("s")
    w = s * _NC + c
    pltpu.sync_copy(zr_hbm, accsh.at[pl.ds(s * _RPW, _RPW)])
    pltpu.sync_copy(ones_hbm, onesv)
    pltpu.sync_copy(dst_hbm.at[w], didx3)
    plsc.subcore_barrier()

    def body(i, _):
        pltpu.sync_copy(onesv, accsh.at[didx3.at[i, 0]], add=True)
        return _

    lax.fori_loop(0, _NCHUNK, body, None)
    plsc.subcore_barrier()
    off2 = pl.multiple_of(c * _N_PAD + s * _RPW, _RPW)
    pltpu.sync_copy(accsh.at[pl.ds(s * _RPW, _RPW)], out_hbm.at[pl.ds(off2, _RPW)])


_sc_params = pltpu.CompilerParams(use_tc_tiling_on_sc=False)

_deg_kernel = functools.partial(
    pl.kernel,
    out_type=jax.ShapeDtypeStruct((_NC * _N_PAD, 8), jnp.float32),
    mesh=_mesh,
    compiler_params=_sc_params,
    scratch_types=[
        pltpu.VMEM_SHARED((_N_PAD, 8), jnp.float32),
        pltpu.VMEM((_NCHUNK, 1, _CHUNK), jnp.int32),
        pltpu.VMEM((_CHUNK, 8), jnp.float32),
        pltpu.SemaphoreType.DMA,
    ],
)(_deg_body)


def _make_agg(dout, chunk):
    nchunk = _EPW // chunk
    stage = dout <= 64

    def _ix(r, j):
        return r.at[j, 0]

    def _agg_body(gh_hbm, src_hbm, dst_hbm, zr_hbm, out_hbm, accsh, sidx3,
                  didx3, rows2, gsems, ssems, *maybe_ghs):
        c = lax.axis_index("c")
        s = lax.axis_index("s")
        w = s * _NC + c
        d0 = pltpu.async_copy(zr_hbm, accsh.at[pl.ds(s * _RPW, _RPW)],
                              gsems.at[0])
        d1 = pltpu.async_copy(src_hbm.at[w], sidx3, gsems.at[1])
        d2 = pltpu.async_copy(dst_hbm.at[w], didx3, gsems.at[2])
        if stage:
            ghs = maybe_ghs[0]
            d3 = pltpu.async_copy(gh_hbm.at[pl.ds(s * _RPW, _RPW)],
                                  ghs.at[pl.ds(s * _RPW, _RPW)], ssems.at[0])
            d3.wait()
            gh_src = ghs
        else:
            gh_src = gh_hbm
        d0.wait()
        d1.wait()
        d2.wait()
        plsc.subcore_barrier()

        pltpu.async_copy(gh_src.at[_ix(sidx3, 0)], rows2.at[0], gsems.at[0])
        pltpu.async_copy(gh_src.at[_ix(sidx3, 1)], rows2.at[1], gsems.at[1])

        def body(j, _):
            p = lax.rem(j, 3)

            @pl.when(j >= 1)
            def _():
                q = lax.rem(j - 1, 3)
                pltpu.make_async_copy(rows2.at[q],
                                      accsh.at[_ix(didx3, j - 1)],
                                      ssems.at[q]).wait()

            @pl.when(j + 2 < nchunk)
            def _():
                q = lax.rem(j + 2, 3)
                pltpu.async_copy(gh_src.at[_ix(sidx3, j + 2)], rows2.at[q],
                                 gsems.at[q])

            pltpu.make_async_copy(gh_src.at[_ix(sidx3, j)], rows2.at[p],
                                  gsems.at[p]).wait()
            pltpu.async_copy(rows2.at[p], accsh.at[_ix(didx3, j)],
                             ssems.at[p], add=True)
            return _

        lax.fori_loop(0, nchunk, body, None)
        q = (nchunk - 1) % 3
        pltpu.make_async_copy(rows2.at[q], accsh.at[_ix(didx3, nchunk - 1)],
                              ssems.at[q]).wait()
        plsc.subcore_barrier()
        off2 = pl.multiple_of(c * _N_PAD + s * _RPW, _RPW)
        pltpu.sync_copy(accsh.at[pl.ds(s * _RPW, _RPW)],
                        out_hbm.at[pl.ds(off2, _RPW)])

    return functools.partial(
        pl.kernel,
        out_type=jax.ShapeDtypeStruct((_NC * _N_PAD, dout), jnp.float32),
        mesh=_mesh,
        compiler_params=_sc_params,
        scratch_types=[
            pltpu.VMEM_SHARED((_N_PAD, dout), jnp.float32),
            pltpu.VMEM((nchunk, 1, chunk), jnp.int32),
            pltpu.VMEM((nchunk, 1, chunk), jnp.int32),
            pltpu.VMEM((3, chunk, dout), jnp.float32),
            pltpu.SemaphoreType.DMA((3,)),
            pltpu.SemaphoreType.DMA((3,)),
        ] + ([pltpu.VMEM_SHARED((_N_PAD, dout), jnp.float32)] if stage else []),
    )(_agg_body)


_agg_chunk = {64: 128, 32: 256, 16: 256, 8: 256}
_agg_kernels = {d: _make_agg(d, _agg_chunk[d]) for d in _agg_chunk}


def _tc0_body(degp_ref, x_ref, w_ref, gh_ref, dinv_ref):
    deg = degp_ref[0, :, 0:1] + degp_ref[1, :, 0:1] + 1.0
    dinv = lax.rsqrt(deg)
    dinv_ref[...] = dinv
    gh_ref[...] = dinv * jnp.dot(x_ref[...], w_ref[...],
                                 preferred_element_type=jnp.float32)


def _tc0(degp, x_pad, w1):
    d1 = _DIMS[1]
    return pl.pallas_call(
        _tc0_body,
        grid=(_NBLK,),
        in_specs=[
            pl.BlockSpec((2, _BLK, 8), lambda i: (0, i, 0)),
            pl.BlockSpec((_BLK, _DIMS[0]), lambda i: (i, 0)),
            pl.BlockSpec((_DIMS[0], d1), lambda i: (0, 0)),
        ],
        out_specs=[
            pl.BlockSpec((_BLK, d1), lambda i: (i, 0)),
            pl.BlockSpec((_BLK, 1), lambda i: (i, 0)),
        ],
        out_shape=[
            jax.ShapeDtypeStruct((_N_PAD, d1), jnp.float32),
            jax.ShapeDtypeStruct((_N_PAD, 1), jnp.float32),
        ],
    )(degp.reshape(2, _N_PAD, 8), x_pad, w1)


def _tcmid_body(a0_ref, a1_ref, gh_ref, dinv_ref, b_ref, w_ref, out_ref):
    i = pl.program_id(0)
    dinv = dinv_ref[...]
    h = jnp.maximum(dinv * (a0_ref[...] + a1_ref[...] + gh_ref[...]) + b_ref[...], 0.0)
    rowid = i * _BLK + lax.broadcasted_iota(jnp.int32, (_BLK, 1), 0)
    h = jnp.where(rowid < _N, h, 0.0)
    out_ref[...] = dinv * jnp.dot(h, w_ref[...], preferred_element_type=jnp.float32)


def _tcmid(accflat, gh, dinv, b2d, wnext, din, dnext):
    return pl.pallas_call(
        _tcmid_body,
        grid=(_NBLK,),
        in_specs=[
            pl.BlockSpec((_BLK, din), lambda i: (i, 0)),
            pl.BlockSpec((_BLK, din), lambda i: (i + _NBLK, 0)),
            pl.BlockSpec((_BLK, din), lambda i: (i, 0)),
            pl.BlockSpec((_BLK, 1), lambda i: (i, 0)),
            pl.BlockSpec((1, din), lambda i: (0, 0)),
            pl.BlockSpec((din, dnext), lambda i: (0, 0)),
        ],
        out_specs=pl.BlockSpec((_BLK, dnext), lambda i: (i, 0)),
        out_shape=jax.ShapeDtypeStruct((_N_PAD, dnext), jnp.float32),
    )(accflat, accflat, gh, dinv, b2d, wnext)


def _tcmid_nomm_body(a0_ref, a1_ref, gh_ref, dinv_ref, b_ref, out_ref):
    i = pl.program_id(0)
    dinv = dinv_ref[...]
    h = jnp.maximum(dinv * (a0_ref[...] + a1_ref[...] + gh_ref[...]) + b_ref[...], 0.0)
    rowid = i * _BLK + lax.broadcasted_iota(jnp.int32, (_BLK, 1), 0)
    h = jnp.where(rowid < _N, h, 0.0)
    out_ref[...] = dinv * h


def _tcmid_nomm(accflat, gh, dinv, b2d, din):
    return pl.pallas_call(
        _tcmid_nomm_body,
        grid=(_NBLK,),
        in_specs=[
            pl.BlockSpec((_BLK, din), lambda i: (i, 0)),
            pl.BlockSpec((_BLK, din), lambda i: (i + _NBLK, 0)),
            pl.BlockSpec((_BLK, din), lambda i: (i, 0)),
            pl.BlockSpec((_BLK, 1), lambda i: (i, 0)),
            pl.BlockSpec((1, din), lambda i: (0, 0)),
        ],
        out_specs=pl.BlockSpec((_BLK, din), lambda i: (i, 0)),
        out_shape=jax.ShapeDtypeStruct((_N_PAD, din), jnp.float32),
    )(accflat, accflat, gh, dinv, b2d)


def _tc5_body(a0_ref, a1_ref, ghp_ref, dinv_ref, w_ref, b_ref, batch_ref,
              out_ref, sums_scr, cnt_scr):
    i = pl.program_id(0)

    @pl.when(i == 0)
    def _():
        sums_scr[...] = jnp.zeros_like(sums_scr)
        cnt_scr[...] = jnp.zeros_like(cnt_scr)

    dinv = dinv_ref[...]
    t = dinv * (a0_ref[...] + a1_ref[...] + ghp_ref[...])
    h = jnp.maximum(jnp.dot(t, w_ref[...], preferred_element_type=jnp.float32)
                    + b_ref[...], 0.0)
    rowid = i * _BLK + lax.broadcasted_iota(jnp.int32, (_BLK, 1), 0)
    h = jnp.where(rowid < _N, h, 0.0)
    gids = lax.broadcasted_iota(jnp.int32, (_BLK, _G), 1)
    mask = (batch_ref[...] == gids).astype(jnp.float32)
    dn = (((0,), (0,)), ((), ()))
    sums_scr[...] += lax.dot_general(mask, h, dn, preferred_element_type=jnp.float32)
    ones = jnp.ones((_BLK, 1), jnp.float32)
    cnt_scr[...] += lax.dot_general(mask, ones, dn, preferred_element_type=jnp.float32)

    @pl.when(i == _NBLK - 1)
    def _():
        out_ref[...] = sums_scr[...] / jnp.maximum(cnt_scr[...], 1.0)


def _tc5(accflat, ghp, dinv, w5, b2d, batch2d):
    din, d5 = _DIMS[4], _DIMS[5]
    return pl.pallas_call(
        _tc5_body,
        grid=(_NBLK,),
        in_specs=[
            pl.BlockSpec((_BLK, din), lambda i: (i, 0)),
            pl.BlockSpec((_BLK, din), lambda i: (i + _NBLK, 0)),
            pl.BlockSpec((_BLK, din), lambda i: (i, 0)),
            pl.BlockSpec((_BLK, 1), lambda i: (i, 0)),
            pl.BlockSpec((din, d5), lambda i: (0, 0)),
            pl.BlockSpec((1, d5), lambda i: (0, 0)),
            pl.BlockSpec((_BLK, 1), lambda i: (i, 0)),
        ],
        out_specs=pl.BlockSpec((_G, d5), lambda i: (0, 0)),
        out_shape=jax.ShapeDtypeStruct((_G, d5), jnp.float32),
        scratch_shapes=[
            pltpu.VMEM((_G, d5), jnp.float32),
            pltpu.VMEM((_G, 1), jnp.float32),
        ],
    )(accflat, accflat, ghp, dinv, w5, b2d, batch2d)


def kernel(x, edge_index, batch, W1, b1, W2, b2, W3, b3, W4, b4, W5, b5):
    src = edge_index[0].astype(jnp.int32)
    dst = edge_index[1].astype(jnp.int32)
    pad_e = jnp.full((_E_PAD - _E,), _N, jnp.int32)
    src_f = jnp.concatenate([src, pad_e])
    dst_f = jnp.concatenate([dst, pad_e])
    dst_p = dst_f.reshape(_NW, _NCHUNK, 1, _CHUNK)

    def _eidx(flat, chunk):
        return flat.reshape(_NW, _EPW // chunk, 1, chunk)

    x_pad = jnp.zeros((_N_PAD, _DIMS[0]), jnp.float32).at[:_N].set(x)
    batch2d = jnp.full((_N_PAD, 1), _G, jnp.int32).at[:_N, 0].set(batch.astype(jnp.int32))

    ones8 = jnp.ones((_CHUNK, 8), jnp.float32)
    zr8 = jnp.zeros((_RPW, 8), jnp.float32)

    degp = _deg_kernel(dst_p, ones8, zr8)
    gh, dinv = _tc0(degp, x_pad, W1)

    params = [(W2, b1), (W3, b2), (W4, b3)]
    for l in range(1, 4):
        din, dnext = _DIMS[l], _DIMS[l + 1]
        ch = _agg_chunk[din]
        zr = jnp.zeros((_RPW, din), jnp.float32)
        accflat = _agg_kernels[din](gh, _eidx(src_f, ch), _eidx(dst_f, ch), zr)
        wnext, b = params[l - 1]
        gh = _tcmid(accflat, gh, dinv, b.reshape(1, din), wnext, din, dnext)

    din = _DIMS[4]
    ch = _agg_chunk[din]
    zr = jnp.zeros((_RPW, din), jnp.float32)
    accflat = _agg_kernels[din](gh, _eidx(src_f, ch), _eidx(dst_f, ch), zr)
    ghp = _tcmid_nomm(accflat, gh, dinv, b4.reshape(1, din), din)

    accflat = _agg_kernels[din](ghp, _eidx(src_f, ch), _eidx(dst_f, ch), zr)
    return _tc5(accflat, ghp, dinv, W5, b5.reshape(1, _DIMS[5]), batch2d)
